# baseline (speedup 1.0000x reference)
import functools

import jax
import jax.numpy as jnp
from jax import lax
from jax.experimental import pallas as pl
from jax.experimental.pallas import tpu as pltpu
from jax.experimental.pallas import tpu_sc as plsc

B, C, H, W = 2, 96, 384, 384
QK_DIM = 96
NUM_HEADS = 3
K_SP = 576
TOPK = 64
HEAD_DIM = QK_DIM // NUM_HEADS
SC_SCALE = HEAD_DIM ** (-0.5)
HW = H * W
N = K_SP * TOPK
BN = B * N

NC, NS = 2, 16
NW = NC * NS

_T1 = 512


def _k1_body(x_ref, lnw_ref, lnb_ref, qw_ref, kw_ref, vw_ref, qk_ref, v_ref):
    xt = jnp.transpose(x_ref[0], (1, 0))
    mu = jnp.mean(xt, axis=1, keepdims=True)
    var = jnp.mean((xt - mu) ** 2, axis=1, keepdims=True)
    xn = (xt - mu) * lax.rsqrt(var + 1e-6)
    xn = xn * lnw_ref[:] + lnb_ref[:]
    dn = (((1,), (1,)), ((), ()))
    q = lax.dot_general(xn, qw_ref[:], dn, preferred_element_type=jnp.float32)
    k = lax.dot_general(xn, kw_ref[:], dn, preferred_element_type=jnp.float32)
    v = lax.dot_general(xn, vw_ref[:], dn, preferred_element_type=jnp.float32)
    qk_ref[0] = jnp.concatenate([q, k], axis=1)
    v_ref[0] = v


def _ln_qkv(xf, ln_w2, ln_b2, q_w, k_w, v_w):
    grid = (B, HW // _T1)
    return pl.pallas_call(
        _k1_body,
        grid=grid,
        in_specs=[
            pl.BlockSpec((1, C, _T1), lambda b, t: (b, 0, t)),
            pl.BlockSpec((1, C), lambda b, t: (0, 0)),
            pl.BlockSpec((1, C), lambda b, t: (0, 0)),
            pl.BlockSpec((C, C), lambda b, t: (0, 0)),
            pl.BlockSpec((C, C), lambda b, t: (0, 0)),
            pl.BlockSpec((C, C), lambda b, t: (0, 0)),
        ],
        out_specs=[
            pl.BlockSpec((1, _T1, 2 * C), lambda b, t: (b, t, 0)),
            pl.BlockSpec((1, _T1, C), lambda b, t: (b, t, 0)),
        ],
        out_shape=[
            jax.ShapeDtypeStruct((B, HW, 2 * C), jnp.float32),
            jax.ShapeDtypeStruct((B, HW, C), jnp.float32),
        ],
        compiler_params=pltpu.CompilerParams(
            dimension_semantics=("parallel", "parallel")),
    )(xf, ln_w2, ln_b2, q_w, k_w, v_w)


_G_ROWS = BN // NW
_G_CH = 96


def _k2_body(qk_hbm, v_hbm, gidx_hbm, qkg_hbm, vg_hbm, idx_v, qbuf, vbuf, sem):
    wid = lax.axis_index("s") * NC + lax.axis_index("c")
    base = wid * _G_ROWS
    pltpu.sync_copy(gidx_hbm.at[pl.ds(base, _G_ROWS)], idx_v)

    def chunk(i, carry):
        off = i * _G_CH
        pltpu.async_copy(qk_hbm.at[idx_v.at[pl.ds(off, _G_CH)]], qbuf, sem).wait()
        pltpu.sync_copy(qbuf, qkg_hbm.at[pl.ds(base + off, _G_CH)])
        pltpu.async_copy(v_hbm.at[idx_v.at[pl.ds(off, _G_CH)]], vbuf, sem).wait()
        pltpu.sync_copy(vbuf, vg_hbm.at[pl.ds(base + off, _G_CH)])
        return carry

    lax.fori_loop(0, _G_ROWS // _G_CH, chunk, 0)


def _sc_gather(qk_tab, v_tab, gidx):
    mesh = plsc.VectorSubcoreMesh(core_axis_name="c", subcore_axis_name="s")
    f = pl.kernel(
        _k2_body,
        out_type=[
            jax.ShapeDtypeStruct((BN, 2 * C), jnp.float32),
            jax.ShapeDtypeStruct((BN, C), jnp.float32),
        ],
        mesh=mesh,
        scratch_types=[
            pltpu.VMEM((_G_ROWS,), jnp.int32),
            pltpu.VMEM((_G_CH, 2 * C), jnp.float32),
            pltpu.VMEM((_G_CH, C), jnp.float32),
            pltpu.SemaphoreType.DMA,
        ],
        compiler_params=pltpu.CompilerParams(use_tc_tiling_on_sc=False),
    )
    return f(qk_tab, v_tab, gidx)


def _k3_body(qk_ref, v_ref, sims_ref, out_ref):
    xqk = qk_ref[0]
    xv = v_ref[0]
    s = jnp.transpose(sims_ref[0], (1, 0))
    outs = []
    for h in range(NUM_HEADS):
        q = xqk[:, h * HEAD_DIM:(h + 1) * HEAD_DIM]
        k = xqk[:, C + h * HEAD_DIM:C + (h + 1) * HEAD_DIM]
        v = xv[:, h * HEAD_DIM:(h + 1) * HEAD_DIM]
        qn = jnp.sum(q * q, axis=1, keepdims=True)
        kn = jnp.sum(k * k, axis=1, keepdims=True)
        dn = (((1,), (1,)), ((), ()))
        qk = lax.dot_general(q, k, dn, preferred_element_type=jnp.float32)
        d2 = qn + jnp.transpose(kn, (1, 0)) - 2.0 * qk
        dist = jnp.sqrt(jnp.maximum(d2, 1e-12))
        a = -SC_SCALE * dist
        a = a - jnp.max(a, axis=1, keepdims=True)
        e = jnp.exp(a)
        attn = e / jnp.sum(e, axis=1, keepdims=True)
        vw = v * s
        dn2 = (((1,), (0,)), ((), ()))
        o = lax.dot_general(attn, vw, dn2, preferred_element_type=jnp.float32)
        outs.append(o * s)
    out_ref[0] = jnp.concatenate(outs, axis=1)


def _attention(qk_g, v_g, sims3):
    grid = (B * K_SP,)
    return pl.pallas_call(
        _k3_body,
        grid=grid,
        in_specs=[
            pl.BlockSpec((1, TOPK, 2 * C), lambda i: (i, 0, 0)),
            pl.BlockSpec((1, TOPK, C), lambda i: (i, 0, 0)),
            pl.BlockSpec((1, 1, TOPK), lambda i: (i, 0, 0)),
        ],
        out_specs=pl.BlockSpec((1, TOPK, C), lambda i: (i, 0, 0)),
        out_shape=jax.ShapeDtypeStruct((B * K_SP, TOPK, C), jnp.float32),
        compiler_params=pltpu.CompilerParams(
            dimension_semantics=("parallel",)),
    )(qk_g, v_g, sims3)


_R = 12288
_NRANGE = HW // _R
_S_TOK = N // NS
_S_CH = 128
_ZROWS = 256


def _k4_body(tok_hbm, gidx_hbm, acc_hbm, cnt_hbm,
             idx_v, idxe_v, buf, ones_v, zbuf, zcnt, acc_sp, cnt_sp):
    c = lax.axis_index("c")
    s = lax.axis_index("s")
    base_tok = c * N + s * _S_TOK

    pltpu.sync_copy(gidx_hbm.at[pl.ds(base_tok, _S_TOK)], idx_v)

    def fill(i, carry):
        ones_v[pl.ds(i * 16, 16)] = jnp.full((16,), 1.0, jnp.float32)
        zcnt[pl.ds(i * 16, 16)] = jnp.zeros((16,), jnp.float32)
        return carry
    lax.fori_loop(0, _S_TOK // 16, fill, 0)

    def zfill(i, carry):
        def zf2(j, carry2):
            zbuf[i, pl.ds(j * 16, 16)] = jnp.zeros((16,), jnp.float32)
            return carry2
        lax.fori_loop(0, C // 16, zf2, 0)
        return carry
    lax.fori_loop(0, _ZROWS, zfill, 0)

    def zc(i, carry):
        pltpu.sync_copy(zcnt, cnt_sp.at[pl.ds((s * 4 + i) * _S_TOK, _S_TOK)])
        return carry
    lax.fori_loop(0, 4, zc, 0)
    plsc.subcore_barrier()

    def cnt_chunk(i, carry):
        def tl(j, carry2):
            vv = idx_v[pl.ds(i * _S_CH + j * 16, 16)] - c * HW
            idxe_v[pl.ds(j * 16, 16)] = vv
            return carry2
        lax.fori_loop(0, _S_CH // 16, tl, 0)
        pltpu.sync_copy(ones_v.at[pl.ds(0, _S_CH)], cnt_sp.at[idxe_v], add=True)
        return carry
    lax.fori_loop(0, _S_TOK // _S_CH, cnt_chunk, 0)
    plsc.subcore_barrier()

    pltpu.sync_copy(cnt_sp.at[pl.ds(s * (HW // NS), HW // NS)],
                    cnt_hbm.at[pl.ds(c * HW + s * (HW // NS), HW // NS)])

    def one_range(rr, carry):
        r0g = c * HW + rr * _R
        def za(i, carry2):
            pltpu.sync_copy(zbuf, acc_sp.at[pl.ds(s * (_R // NS) + i * _ZROWS,
                                                  _ZROWS)])
            return carry2
        lax.fori_loop(0, _R // NS // _ZROWS, za, 0)
        plsc.subcore_barrier()

        def chunk(i, carry2):
            def tl(j, carry3):
                vv = idx_v[pl.ds(i * _S_CH + j * 16, 16)] - r0g
                oob = (vv < 0) | (vv >= _R)
                vv = jnp.where(oob, _R, vv)
                idxe_v[pl.ds(j * 16, 16)] = vv
                return carry3
            lax.fori_loop(0, _S_CH // 16, tl, 0)
            pltpu.sync_copy(tok_hbm.at[pl.ds(base_tok + i * _S_CH, _S_CH)], buf)
            pltpu.sync_copy(buf, acc_sp.at[idxe_v], add=True)
            return carry2
        lax.fori_loop(0, _S_TOK // _S_CH, chunk, 0)
        plsc.subcore_barrier()

        pltpu.sync_copy(acc_sp.at[pl.ds(s * (_R // NS), _R // NS)],
                        acc_hbm.at[pl.ds(r0g + s * (_R // NS), _R // NS)])
        plsc.subcore_barrier()
        return carry

    lax.fori_loop(0, _NRANGE, one_range, 0)


def _sc_scatter(out_tok, gidx):
    mesh = plsc.VectorSubcoreMesh(core_axis_name="c", subcore_axis_name="s")
    f = pl.kernel(
        _k4_body,
        out_type=[
            jax.ShapeDtypeStruct((B * HW, C), jnp.float32),
            jax.ShapeDtypeStruct((B * HW,), jnp.float32),
        ],
        mesh=mesh,
        scratch_types=[
            pltpu.VMEM((_S_TOK,), jnp.int32),
            pltpu.VMEM((_S_CH,), jnp.int32),
            pltpu.VMEM((_S_CH, C), jnp.float32),
            pltpu.VMEM((_S_TOK,), jnp.float32),
            pltpu.VMEM((_ZROWS, C), jnp.float32),
            pltpu.VMEM((_S_TOK,), jnp.float32),
            pltpu.VMEM_SHARED((_R + 16, C), jnp.float32),
            pltpu.VMEM_SHARED((HW + 16,), jnp.float32),
        ],
        compiler_params=pltpu.CompilerParams(use_tc_tiling_on_sc=False),
    )
    return f(out_tok, gidx)


def _k5_body(acc_ref, cnt_ref, v_ref, out_ref):
    a = acc_ref[0]
    ct = jnp.transpose(cnt_ref[0, 0], (1, 0))
    v = v_ref[0]
    mean = a / jnp.maximum(ct, 1.0)
    res = jnp.where(ct > 1e-5, mean, v)
    out_ref[0] = jnp.transpose(res, (1, 0))


def _merge(acc4, cnt4, v_tok):
    grid = (B, HW // _T1)
    return pl.pallas_call(
        _k5_body,
        grid=grid,
        in_specs=[
            pl.BlockSpec((1, _T1, C), lambda b, t: (b, t, 0)),
            pl.BlockSpec((1, 1, 1, _T1), lambda b, t: (b, t, 0, 0)),
            pl.BlockSpec((1, _T1, C), lambda b, t: (b, t, 0)),
        ],
        out_specs=pl.BlockSpec((1, C, _T1), lambda b, t: (b, 0, t)),
        out_shape=jax.ShapeDtypeStruct((B, C, HW), jnp.float32),
        compiler_params=pltpu.CompilerParams(
            dimension_semantics=("parallel", "parallel")),
    )(acc4, cnt4, v_tok)


@jax.jit
def _run(x, sims, ln_w, ln_b, q_w, k_w, v_w, indices):
    xf = x.reshape(B, C, HW)
    qk_tok, v_tok = _ln_qkv(xf, ln_w.reshape(1, C), ln_b.reshape(1, C),
                            q_w, k_w, v_w)
    gidx = (indices.reshape(B, N)
            + (jnp.arange(B, dtype=jnp.int32) * HW)[:, None]).reshape(BN)
    qk_g, v_g = _sc_gather(qk_tok.reshape(B * HW, 2 * C),
                           v_tok.reshape(B * HW, C), gidx)
    out_tok = _attention(qk_g.reshape(B * K_SP, TOPK, 2 * C),
                         v_g.reshape(B * K_SP, TOPK, C),
                         sims.reshape(B * K_SP, 1, TOPK))
    acc, cnt = _sc_scatter(out_tok.reshape(BN, C), gidx)
    out = _merge(acc.reshape(B, HW, C),
                 cnt.reshape(B, HW // _T1, 1, _T1),
                 v_tok)
    return out.reshape(B, C, H, W)


def kernel(x, sims, mask, ln_w, ln_b, q_w, k_w, v_w, indices, labels,
           num_spixels):
    del mask, labels, num_spixels
    return _run(x, sims, ln_w, ln_b, q_w, k_w, v_w, indices)


# speedup vs baseline: 1.0495x; 1.0495x over previous
import functools

import jax
import jax.numpy as jnp
from jax import lax
from jax.experimental import pallas as pl
from jax.experimental.pallas import tpu as pltpu
from jax.experimental.pallas import tpu_sc as plsc

B, C, H, W = 2, 96, 384, 384
QK_DIM = 96
NUM_HEADS = 3
K_SP = 576
TOPK = 64
HEAD_DIM = QK_DIM // NUM_HEADS
SC_SCALE = HEAD_DIM ** (-0.5)
HW = H * W
N = K_SP * TOPK
BN = B * N

NC, NS = 2, 16
NW = NC * NS

_T1 = 512


def _k1_body(x_ref, lnw_ref, lnb_ref, qw_ref, kw_ref, vw_ref, qk_ref, v_ref):
    xt = jnp.transpose(x_ref[0], (1, 0))
    mu = jnp.mean(xt, axis=1, keepdims=True)
    var = jnp.mean((xt - mu) ** 2, axis=1, keepdims=True)
    xn = (xt - mu) * lax.rsqrt(var + 1e-6)
    xn = xn * lnw_ref[:] + lnb_ref[:]
    dn = (((1,), (1,)), ((), ()))
    q = lax.dot_general(xn, qw_ref[:], dn, preferred_element_type=jnp.float32)
    k = lax.dot_general(xn, kw_ref[:], dn, preferred_element_type=jnp.float32)
    v = lax.dot_general(xn, vw_ref[:], dn, preferred_element_type=jnp.float32)
    qk_ref[0] = jnp.concatenate([q, k], axis=1)
    v_ref[0] = v


def _ln_qkv(xf, ln_w2, ln_b2, q_w, k_w, v_w):
    grid = (B, HW // _T1)
    return pl.pallas_call(
        _k1_body,
        grid=grid,
        in_specs=[
            pl.BlockSpec((1, C, _T1), lambda b, t: (b, 0, t)),
            pl.BlockSpec((1, C), lambda b, t: (0, 0)),
            pl.BlockSpec((1, C), lambda b, t: (0, 0)),
            pl.BlockSpec((C, C), lambda b, t: (0, 0)),
            pl.BlockSpec((C, C), lambda b, t: (0, 0)),
            pl.BlockSpec((C, C), lambda b, t: (0, 0)),
        ],
        out_specs=[
            pl.BlockSpec((1, _T1, 2 * C), lambda b, t: (b, t, 0)),
            pl.BlockSpec((1, _T1, C), lambda b, t: (b, t, 0)),
        ],
        out_shape=[
            jax.ShapeDtypeStruct((B, HW, 2 * C), jnp.float32),
            jax.ShapeDtypeStruct((B, HW, C), jnp.float32),
        ],
        compiler_params=pltpu.CompilerParams(
            dimension_semantics=("parallel", "parallel")),
    )(xf, ln_w2, ln_b2, q_w, k_w, v_w)


_G_ROWS = BN // NW
_G_CH = 96


def _k2_body(qk_hbm, v_hbm, gidx_hbm, qkg_hbm, vg_hbm, idx_v, qbuf, vbuf, sem):
    wid = lax.axis_index("s") * NC + lax.axis_index("c")
    base = wid * _G_ROWS
    pltpu.sync_copy(gidx_hbm.at[pl.ds(base, _G_ROWS)], idx_v)

    def chunk(i, carry):
        off = i * _G_CH
        pltpu.async_copy(qk_hbm.at[idx_v.at[pl.ds(off, _G_CH)]], qbuf, sem).wait()
        pltpu.sync_copy(qbuf, qkg_hbm.at[pl.ds(base + off, _G_CH)])
        pltpu.async_copy(v_hbm.at[idx_v.at[pl.ds(off, _G_CH)]], vbuf, sem).wait()
        pltpu.sync_copy(vbuf, vg_hbm.at[pl.ds(base + off, _G_CH)])
        return carry

    lax.fori_loop(0, _G_ROWS // _G_CH, chunk, 0)


def _sc_gather(qk_tab, v_tab, gidx):
    mesh = plsc.VectorSubcoreMesh(core_axis_name="c", subcore_axis_name="s")
    f = pl.kernel(
        _k2_body,
        out_type=[
            jax.ShapeDtypeStruct((BN, 2 * C), jnp.float32),
            jax.ShapeDtypeStruct((BN, C), jnp.float32),
        ],
        mesh=mesh,
        scratch_types=[
            pltpu.VMEM((_G_ROWS,), jnp.int32),
            pltpu.VMEM((_G_CH, 2 * C), jnp.float32),
            pltpu.VMEM((_G_CH, C), jnp.float32),
            pltpu.SemaphoreType.DMA,
        ],
        compiler_params=pltpu.CompilerParams(use_tc_tiling_on_sc=False),
    )
    return f(qk_tab, v_tab, gidx)


_GSP = 8


def _k3_body(qk_ref, v_ref, sims_ref, out_ref):
    xqk = qk_ref[0]
    xv = v_ref[0]
    sm = sims_ref[0]
    blocks = []
    for g in range(_GSP):
        r0 = g * TOPK
        s = jnp.transpose(sm[g:g + 1, :], (1, 0))
        outs = []
        for h in range(NUM_HEADS):
            q = xqk[r0:r0 + TOPK, h * HEAD_DIM:(h + 1) * HEAD_DIM]
            k = xqk[r0:r0 + TOPK, C + h * HEAD_DIM:C + (h + 1) * HEAD_DIM]
            v = xv[r0:r0 + TOPK, h * HEAD_DIM:(h + 1) * HEAD_DIM]
            qn = jnp.sum(q * q, axis=1, keepdims=True)
            kn = jnp.sum(k * k, axis=1, keepdims=True)
            dn = (((1,), (1,)), ((), ()))
            qk = lax.dot_general(q, k, dn, preferred_element_type=jnp.float32)
            d2 = qn + jnp.transpose(kn, (1, 0)) - 2.0 * qk
            dist = jnp.sqrt(jnp.maximum(d2, 1e-12))
            a = -SC_SCALE * dist
            a = a - jnp.max(a, axis=1, keepdims=True)
            e = jnp.exp(a)
            attn = e / jnp.sum(e, axis=1, keepdims=True)
            vw = v * s
            dn2 = (((1,), (0,)), ((), ()))
            o = lax.dot_general(attn, vw, dn2,
                                preferred_element_type=jnp.float32)
            outs.append(o * s)
        blocks.append(jnp.concatenate(outs, axis=1))
    out_ref[0] = jnp.concatenate(blocks, axis=0)


def _attention(qk_g, v_g, sims3):
    grid = (B * K_SP // _GSP,)
    return pl.pallas_call(
        _k3_body,
        grid=grid,
        in_specs=[
            pl.BlockSpec((1, _GSP * TOPK, 2 * C), lambda i: (i, 0, 0)),
            pl.BlockSpec((1, _GSP * TOPK, C), lambda i: (i, 0, 0)),
            pl.BlockSpec((1, _GSP, TOPK), lambda i: (i, 0, 0)),
        ],
        out_specs=pl.BlockSpec((1, _GSP * TOPK, C), lambda i: (i, 0, 0)),
        out_shape=jax.ShapeDtypeStruct((B * K_SP // _GSP, _GSP * TOPK, C),
                                       jnp.float32),
        compiler_params=pltpu.CompilerParams(
            dimension_semantics=("parallel",)),
    )(qk_g, v_g, sims3)


_R = 12288
_NRANGE = HW // _R
_S_TOK = N // NS
_S_CH = 128
_ZROWS = 64


_NCH = _S_TOK // _S_CH


def _k4_body(tok_hbm, gidx_hbm, acc_hbm, cnt_hbm,
             idx_v, idxl_v, idxe2d, bufa, bufb, ones_v, zbuf, zcnt,
             sema, semb, acc_sp, cnt_sp):
    c = lax.axis_index("c")
    s = lax.axis_index("s")
    base_tok = c * N + s * _S_TOK

    pltpu.sync_copy(gidx_hbm.at[pl.ds(base_tok, _S_TOK)], idx_v)

    def fill(i, carry):
        ones_v[pl.ds(i * 16, 16)] = jnp.full((16,), 1.0, jnp.float32)
        zcnt[pl.ds(i * 16, 16)] = jnp.zeros((16,), jnp.float32)
        idxl_v[pl.ds(i * 16, 16)] = idx_v[pl.ds(i * 16, 16)] - c * HW
        return carry
    lax.fori_loop(0, _S_TOK // 16, fill, 0)

    def zfill(i, carry):
        def zf2(j, carry2):
            zbuf[i, pl.ds(j * 16, 16)] = jnp.zeros((16,), jnp.float32)
            return carry2
        lax.fori_loop(0, C // 16, zf2, 0)
        return carry
    lax.fori_loop(0, _ZROWS, zfill, 0)

    def zc(i, carry):
        pltpu.sync_copy(zcnt, cnt_sp.at[pl.ds((s * 4 + i) * _S_TOK, _S_TOK)])
        return carry
    lax.fori_loop(0, 4, zc, 0)
    plsc.subcore_barrier()
    pltpu.sync_copy(ones_v, cnt_sp.at[idxl_v], add=True)
    plsc.subcore_barrier()
    pltpu.sync_copy(cnt_sp.at[pl.ds(s * (HW // NS), HW // NS)],
                    cnt_hbm.at[pl.ds(c * HW + s * (HW // NS), HW // NS)])

    def start(buf, i):
        return pltpu.async_copy(
            tok_hbm.at[pl.ds(base_tok + i * _S_CH, _S_CH)], buf,
            sema if buf is bufa else semb)

    def wait(buf):
        pltpu.make_async_copy(
            tok_hbm.at[pl.ds(base_tok, _S_CH)], buf,
            sema if buf is bufa else semb).wait()

    def one_range(rr, carry):
        r0g = c * HW + rr * _R
        def za(i, carry2):
            pltpu.sync_copy(zbuf, acc_sp.at[pl.ds(s * (_R // NS) + i * _ZROWS,
                                                  _ZROWS)])
            return carry2
        lax.fori_loop(0, _R // NS // _ZROWS, za, 0)

        def tl(i, carry2):
            def tl2(j, carry3):
                vv = idx_v[pl.ds(i * _S_CH + j * 16, 16)] - r0g
                oob = (vv < 0) | (vv >= _R)
                vv = jnp.where(oob, _R, vv)
                idxe2d[i, pl.ds(j * 16, 16)] = vv
                return carry3
            lax.fori_loop(0, _S_CH // 16, tl2, 0)
            return carry2
        lax.fori_loop(0, _NCH, tl, 0)
        plsc.subcore_barrier()

        start(bufa, 0)

        def chunk(i, carry2):
            i2 = i * 2
            start(bufb, i2 + 1)
            wait(bufa)
            pltpu.sync_copy(bufa, acc_sp.at[idxe2d.at[i2]], add=True)

            @pl.when(i2 + 2 < _NCH)
            def _():
                start(bufa, i2 + 2)
            wait(bufb)
            pltpu.sync_copy(bufb, acc_sp.at[idxe2d.at[i2 + 1]], add=True)
            return carry2
        lax.fori_loop(0, _NCH // 2, chunk, 0)
        plsc.subcore_barrier()

        pltpu.sync_copy(acc_sp.at[pl.ds(s * (_R // NS), _R // NS)],
                        acc_hbm.at[pl.ds(r0g + s * (_R // NS), _R // NS)])
        plsc.subcore_barrier()
        return carry

    lax.fori_loop(0, _NRANGE, one_range, 0)


def _sc_scatter(out_tok, gidx):
    mesh = plsc.VectorSubcoreMesh(core_axis_name="c", subcore_axis_name="s")
    f = pl.kernel(
        _k4_body,
        out_type=[
            jax.ShapeDtypeStruct((B * HW, C), jnp.float32),
            jax.ShapeDtypeStruct((B * HW,), jnp.float32),
        ],
        mesh=mesh,
        scratch_types=[
            pltpu.VMEM((_S_TOK,), jnp.int32),
            pltpu.VMEM((_S_TOK,), jnp.int32),
            pltpu.VMEM((_NCH, _S_CH), jnp.int32),
            pltpu.VMEM((_S_CH, C), jnp.float32),
            pltpu.VMEM((_S_CH, C), jnp.float32),
            pltpu.VMEM((_S_TOK,), jnp.float32),
            pltpu.VMEM((_ZROWS, C), jnp.float32),
            pltpu.VMEM((_S_TOK,), jnp.float32),
            pltpu.SemaphoreType.DMA,
            pltpu.SemaphoreType.DMA,
            pltpu.VMEM_SHARED((_R + 16, C), jnp.float32),
            pltpu.VMEM_SHARED((HW + 16,), jnp.float32),
        ],
        compiler_params=pltpu.CompilerParams(use_tc_tiling_on_sc=False),
    )
    return f(out_tok, gidx)


def _k5_body(acc_ref, cnt_ref, v_ref, out_ref):
    a = acc_ref[0]
    ct = jnp.transpose(cnt_ref[0, 0], (1, 0))
    v = v_ref[0]
    mean = a / jnp.maximum(ct, 1.0)
    res = jnp.where(ct > 1e-5, mean, v)
    out_ref[0] = jnp.transpose(res, (1, 0))


def _merge(acc4, cnt4, v_tok):
    grid = (B, HW // _T1)
    return pl.pallas_call(
        _k5_body,
        grid=grid,
        in_specs=[
            pl.BlockSpec((1, _T1, C), lambda b, t: (b, t, 0)),
            pl.BlockSpec((1, 1, 1, _T1), lambda b, t: (b, t, 0, 0)),
            pl.BlockSpec((1, _T1, C), lambda b, t: (b, t, 0)),
        ],
        out_specs=pl.BlockSpec((1, C, _T1), lambda b, t: (b, 0, t)),
        out_shape=jax.ShapeDtypeStruct((B, C, HW), jnp.float32),
        compiler_params=pltpu.CompilerParams(
            dimension_semantics=("parallel", "parallel")),
    )(acc4, cnt4, v_tok)


@jax.jit
def _run(x, sims, ln_w, ln_b, q_w, k_w, v_w, indices):
    xf = x.reshape(B, C, HW)
    qk_tok, v_tok = _ln_qkv(xf, ln_w.reshape(1, C), ln_b.reshape(1, C),
                            q_w, k_w, v_w)
    gidx = (indices.reshape(B, N)
            + (jnp.arange(B, dtype=jnp.int32) * HW)[:, None]).reshape(BN)
    qk_g, v_g = _sc_gather(qk_tok.reshape(B * HW, 2 * C),
                           v_tok.reshape(B * HW, C), gidx)
    ng = B * K_SP // _GSP
    out_tok = _attention(qk_g.reshape(ng, _GSP * TOPK, 2 * C),
                         v_g.reshape(ng, _GSP * TOPK, C),
                         sims.reshape(ng, _GSP, TOPK))
    acc, cnt = _sc_scatter(out_tok.reshape(BN, C), gidx)
    out = _merge(acc.reshape(B, HW, C),
                 cnt.reshape(B, HW // _T1, 1, _T1),
                 v_tok)
    return out.reshape(B, C, H, W)


def kernel(x, sims, mask, ln_w, ln_b, q_w, k_w, v_w, indices, labels,
           num_spixels):
    del mask, labels, num_spixels
    return _run(x, sims, ln_w, ln_b, q_w, k_w, v_w, indices)


# speedup vs baseline: 1.6072x; 1.5314x over previous
import jax
import jax.numpy as jnp
from jax import lax
from jax.experimental import pallas as pl
from jax.experimental.pallas import tpu as pltpu
from jax.experimental.pallas import tpu_sc as plsc

B, C, H, W = 2, 96, 384, 384
QK_DIM = 96
NUM_HEADS = 3
K_SP = 576
TOPK = 64
HEAD_DIM = QK_DIM // NUM_HEADS
SC_SCALE = HEAD_DIM ** (-0.5)
HW = H * W
N = K_SP * TOPK
BN = B * N

NC, NS = 2, 16
NW = NC * NS

_HB1 = 8
_T1 = _HB1 * W


def _k1_body(x_ref, lnw_ref, lnb_ref, qw_ref, kw_ref, vw_ref, qk_ref, v_ref):
    x2 = jnp.reshape(x_ref[0], (C, _T1))
    xt = jnp.transpose(x2, (1, 0))
    mu = jnp.mean(xt, axis=1, keepdims=True)
    var = jnp.mean((xt - mu) ** 2, axis=1, keepdims=True)
    xn = (xt - mu) * lax.rsqrt(var + 1e-6)
    xn = xn * lnw_ref[:] + lnb_ref[:]
    dn = (((1,), (1,)), ((), ()))
    q = lax.dot_general(xn, qw_ref[:], dn, preferred_element_type=jnp.float32)
    k = lax.dot_general(xn, kw_ref[:], dn, preferred_element_type=jnp.float32)
    v = lax.dot_general(xn, vw_ref[:], dn, preferred_element_type=jnp.float32)
    qk_ref[...] = jnp.concatenate([q, k], axis=1)
    v_ref[...] = v


def _ln_qkv(x, ln_w2, ln_b2, q_w, k_w, v_w):
    grid = (B, H // _HB1)
    return pl.pallas_call(
        _k1_body,
        grid=grid,
        in_specs=[
            pl.BlockSpec((1, C, _HB1, W), lambda b, t: (b, 0, t, 0)),
            pl.BlockSpec((1, C), lambda b, t: (0, 0)),
            pl.BlockSpec((1, C), lambda b, t: (0, 0)),
            pl.BlockSpec((C, C), lambda b, t: (0, 0)),
            pl.BlockSpec((C, C), lambda b, t: (0, 0)),
            pl.BlockSpec((C, C), lambda b, t: (0, 0)),
        ],
        out_specs=[
            pl.BlockSpec((_T1, 2 * C), lambda b, t: (b * (H // _HB1) + t, 0)),
            pl.BlockSpec((_T1, C), lambda b, t: (b * (H // _HB1) + t, 0)),
        ],
        out_shape=[
            jax.ShapeDtypeStruct((B * HW, 2 * C), jnp.float32),
            jax.ShapeDtypeStruct((B * HW, C), jnp.float32),
        ],
        compiler_params=pltpu.CompilerParams(
            dimension_semantics=("parallel", "parallel")),
    )(x, ln_w2, ln_b2, q_w, k_w, v_w)


_G_ROWS = BN // NW
_G_CH = 96


def _k2_body(qk_hbm, v_hbm, gidx_hbm, qkg_hbm, vg_hbm, idx_v, qbuf, vbuf, sem):
    wid = lax.axis_index("s") * NC + lax.axis_index("c")
    base = wid * _G_ROWS
    pltpu.sync_copy(gidx_hbm.at[pl.ds(base, _G_ROWS)], idx_v)

    def chunk(i, carry):
        off = i * _G_CH
        pltpu.async_copy(qk_hbm.at[idx_v.at[pl.ds(off, _G_CH)]], qbuf, sem).wait()
        pltpu.sync_copy(qbuf, qkg_hbm.at[pl.ds(base + off, _G_CH)])
        pltpu.async_copy(v_hbm.at[idx_v.at[pl.ds(off, _G_CH)]], vbuf, sem).wait()
        pltpu.sync_copy(vbuf, vg_hbm.at[pl.ds(base + off, _G_CH)])
        return carry

    lax.fori_loop(0, _G_ROWS // _G_CH, chunk, 0)


def _sc_gather(qk_tab, v_tab, gidx):
    mesh = plsc.VectorSubcoreMesh(core_axis_name="c", subcore_axis_name="s")
    f = pl.kernel(
        _k2_body,
        out_type=[
            jax.ShapeDtypeStruct((BN, 2 * C), jnp.float32),
            jax.ShapeDtypeStruct((BN, C), jnp.float32),
        ],
        mesh=mesh,
        scratch_types=[
            pltpu.VMEM((_G_ROWS,), jnp.int32),
            pltpu.VMEM((_G_CH, 2 * C), jnp.float32),
            pltpu.VMEM((_G_CH, C), jnp.float32),
            pltpu.SemaphoreType.DMA,
        ],
        compiler_params=pltpu.CompilerParams(use_tc_tiling_on_sc=False),
    )
    return f(qk_tab, v_tab, gidx)


_GSP = 8
_RWS = _GSP * TOPK


def _k3_body(qk_ref, v_ref, sims_ref, out_ref, asc):
    X = qk_ref[...]
    XV = v_ref[...]
    sT = jnp.transpose(sims_ref[0], (1, 0))

    @pl.when(pl.program_id(0) == 0)
    def _():
        asc[...] = jnp.zeros((_RWS, _RWS), jnp.float32)

    outs = []
    for h in range(NUM_HEADS):
        q = X[:, h * HEAD_DIM:(h + 1) * HEAD_DIM]
        k = X[:, C + h * HEAD_DIM:C + (h + 1) * HEAD_DIM]
        v = XV[:, h * HEAD_DIM:(h + 1) * HEAD_DIM]
        qn = jnp.sum(q * q, axis=1, keepdims=True)
        kn = jnp.sum(k * k, axis=1, keepdims=True)
        dn = (((1,), (1,)), ((), ()))
        QK = lax.dot_general(q, k, dn, preferred_element_type=jnp.float32)
        qkd = jnp.concatenate(
            [QK[s * TOPK:(s + 1) * TOPK, s * TOPK:(s + 1) * TOPK]
             for s in range(_GSP)], axis=0)
        knd = jnp.concatenate(
            [jnp.broadcast_to(
                jnp.transpose(kn[s * TOPK:(s + 1) * TOPK], (1, 0)),
                (TOPK, TOPK)) for s in range(_GSP)], axis=0)
        d2 = qn + knd - 2.0 * qkd
        dist = jnp.sqrt(jnp.maximum(d2, 1e-12))
        a = -SC_SCALE * dist
        a = a - jnp.max(a, axis=1, keepdims=True)
        e = jnp.exp(a)
        attn = e / jnp.sum(e, axis=1, keepdims=True)
        for s in range(_GSP):
            asc[s * TOPK:(s + 1) * TOPK, s * TOPK:(s + 1) * TOPK] = (
                attn[s * TOPK:(s + 1) * TOPK, :])
        vw = v * sT
        dn2 = (((1,), (0,)), ((), ()))
        o = lax.dot_general(asc[...], vw, dn2,
                            preferred_element_type=jnp.float32)
        outs.append(o * sT)
    out_ref[...] = jnp.concatenate(outs, axis=1)


def _attention(qk_g, v_g, sims2):
    grid = (BN // _RWS,)
    return pl.pallas_call(
        _k3_body,
        grid=grid,
        in_specs=[
            pl.BlockSpec((_RWS, 2 * C), lambda i: (i, 0)),
            pl.BlockSpec((_RWS, C), lambda i: (i, 0)),
            pl.BlockSpec((1, 1, _RWS), lambda i: (i, 0, 0)),
        ],
        out_specs=pl.BlockSpec((_RWS, C), lambda i: (i, 0)),
        out_shape=jax.ShapeDtypeStruct((BN, C), jnp.float32),
        scratch_shapes=[pltpu.VMEM((_RWS, _RWS), jnp.float32)],
        compiler_params=pltpu.CompilerParams(
            dimension_semantics=("arbitrary",)),
    )(qk_g, v_g, sims2)


_R = 12288
_NRANGE = HW // _R
_S_TOK = N // NS
_S_CH = 128
_NCH = _S_TOK // _S_CH
_ZROWS = 64
_CROWS = HW // NS // 512


def _k4_body(tok_hbm, gidx_hbm, acc_hbm, cnt_hbm,
             idxa_v, idxb_v, idxe2d, bufa, bufb, zbuf, cntv,
             sema, semb, semz, acc_sp):
    c = lax.axis_index("c")
    s = lax.axis_index("s")
    base_tok = c * N + s * _S_TOK
    ones16 = jnp.full((16,), 1.0, jnp.float32)

    with jax.named_scope("k4_init"):
        def zfill(i, carry):
            def zf2(j, carry2):
                zbuf[i, pl.ds(j * 16, 16)] = jnp.zeros((16,), jnp.float32)
                return carry2
            lax.fori_loop(0, C // 16, zf2, 0)
            return carry
        lax.fori_loop(0, _ZROWS, zfill, 0)

        def cfill(i, carry):
            cntv[pl.ds(i * 16, 16)] = jnp.zeros((16,), jnp.float32)
            return carry
        lax.fori_loop(0, (HW // NS) // 16, cfill, 0)

    with jax.named_scope("k4_cnt"):
        pix0 = c * HW + s * (HW // NS)
        pltpu.async_copy(gidx_hbm.at[pl.ds(c * N, _S_TOK)], idxa_v, sema)

        def scan_tile(tt, carry):
            pltpu.make_async_copy(gidx_hbm.at[pl.ds(0, _S_TOK)],
                                  idxa_v, sema).wait()
            def mv(i, carry2):
                idxb_v[pl.ds(i * 16, 16)] = idxa_v[pl.ds(i * 16, 16)]
                return carry2
            lax.fori_loop(0, _S_TOK // 16, mv, 0)

            @pl.when(tt + 1 < NS)
            def _():
                pltpu.async_copy(
                    gidx_hbm.at[pl.ds(c * N + (tt + 1) * _S_TOK, _S_TOK)],
                    idxa_v, sema)

            def scan(i, carry2):
                vv = idxb_v[pl.ds(i * 16, 16)] - pix0
                msk = (vv >= 0) & (vv < HW // NS)
                vv = jnp.where(msk, vv, 0)
                plsc.addupdate_scatter(cntv, [vv], ones16, mask=msk)
                return carry2
            lax.fori_loop(0, _S_TOK // 16, scan, 0)
            return carry
        lax.fori_loop(0, NS, scan_tile, 0)
        pltpu.sync_copy(cntv, cnt_hbm.at[0, pl.ds(pix0, HW // NS)])

    pltpu.sync_copy(gidx_hbm.at[pl.ds(base_tok, _S_TOK)], idxa_v)

    def start(buf, i, sem):
        return pltpu.async_copy(
            tok_hbm.at[pl.ds(base_tok + i * _S_CH, _S_CH)], buf, sem)

    def wait(buf, sem):
        pltpu.make_async_copy(
            tok_hbm.at[pl.ds(base_tok, _S_CH)], buf, sem).wait()

    def one_range(rr, carry):
        r0g = c * HW + rr * _R
        with jax.named_scope("k4_zero"):
            def za(i, carry2):
                pltpu.async_copy(
                    zbuf,
                    acc_sp.at[pl.ds(s * (_R // NS) + i * _ZROWS, _ZROWS)],
                    semz)
                return carry2
            lax.fori_loop(0, _R // NS // _ZROWS, za, 0)

        with jax.named_scope("k4_idxe"):
            def tl(i, carry2):
                def tl2(j, carry3):
                    vv = idxa_v[pl.ds(i * _S_CH + j * 16, 16)] - r0g
                    oob = (vv < 0) | (vv >= _R)
                    vv = jnp.where(oob, _R, vv)
                    idxe2d[i, pl.ds(j * 16, 16)] = vv
                    return carry3
                lax.fori_loop(0, _S_CH // 16, tl2, 0)
                return carry2
            lax.fori_loop(0, _NCH, tl, 0)

        with jax.named_scope("k4_zdrain"):
            def zd(i, carry2):
                pltpu.make_async_copy(
                    zbuf, acc_sp.at[pl.ds(s * (_R // NS), _ZROWS)],
                    semz).wait()
                return carry2
            lax.fori_loop(0, _R // NS // _ZROWS, zd, 0)
        plsc.subcore_barrier()

        with jax.named_scope("k4_chunks"):
            start(bufa, 0, sema)

            def chunk(i, carry2):
                i2 = i * 2
                start(bufb, i2 + 1, semb)
                wait(bufa, sema)
                pltpu.sync_copy(bufa, acc_sp.at[idxe2d.at[i2]], add=True)

                @pl.when(i2 + 2 < _NCH)
                def _():
                    start(bufa, i2 + 2, sema)
                wait(bufb, semb)
                pltpu.sync_copy(bufb, acc_sp.at[idxe2d.at[i2 + 1]], add=True)
                return carry2
            lax.fori_loop(0, _NCH // 2, chunk, 0)
        plsc.subcore_barrier()

        with jax.named_scope("k4_out"):
            pltpu.sync_copy(acc_sp.at[pl.ds(s * (_R // NS), _R // NS)],
                            acc_hbm.at[pl.ds(r0g + s * (_R // NS), _R // NS)])
        plsc.subcore_barrier()
        return carry

    lax.fori_loop(0, _NRANGE, one_range, 0)


def _sc_scatter(out_tok, gidx):
    mesh = plsc.VectorSubcoreMesh(core_axis_name="c", subcore_axis_name="s")
    f = pl.kernel(
        _k4_body,
        out_type=[
            jax.ShapeDtypeStruct((B * HW, C), jnp.float32),
            jax.ShapeDtypeStruct((1, B * HW), jnp.float32),
        ],
        mesh=mesh,
        scratch_types=[
            pltpu.VMEM((_S_TOK,), jnp.int32),
            pltpu.VMEM((_S_TOK,), jnp.int32),
            pltpu.VMEM((_NCH, _S_CH), jnp.int32),
            pltpu.VMEM((_S_CH, C), jnp.float32),
            pltpu.VMEM((_S_CH, C), jnp.float32),
            pltpu.VMEM((_ZROWS, C), jnp.float32),
            pltpu.VMEM((HW // NS,), jnp.float32),
            pltpu.SemaphoreType.DMA,
            pltpu.SemaphoreType.DMA,
            pltpu.SemaphoreType.DMA,
            pltpu.VMEM_SHARED((_R + 16, C), jnp.float32),
        ],
        compiler_params=pltpu.CompilerParams(use_tc_tiling_on_sc=False,
                                             needs_layout_passes=False),
    )
    return f(out_tok, gidx)


_T5 = 12288
_HB5 = _T5 // W


def _k5_body(acc_ref, cnt_ref, v_ref, out_ref):
    a = acc_ref[...]
    ct = jnp.transpose(cnt_ref[...], (1, 0))
    v = v_ref[...]
    mean = a / jnp.maximum(ct, 1.0)
    res = jnp.where(ct > 1e-5, mean, v)
    rT = jnp.transpose(res, (1, 0))
    for hb in range(_HB5):
        out_ref[0, :, hb, :] = rT[:, hb * W:(hb + 1) * W]


def _merge(acc, cnt2, v_tok):
    grid = (B * HW // _T5,)
    nh = H // _HB5
    return pl.pallas_call(
        _k5_body,
        grid=grid,
        in_specs=[
            pl.BlockSpec((_T5, C), lambda t: (t, 0)),
            pl.BlockSpec((1, _T5), lambda t: (0, t)),
            pl.BlockSpec((_T5, C), lambda t: (t, 0)),
        ],
        out_specs=pl.BlockSpec((1, C, _HB5, W), lambda t: (t // nh, 0, t % nh, 0)),
        out_shape=jax.ShapeDtypeStruct((B, C, H, W), jnp.float32),
        compiler_params=pltpu.CompilerParams(
            dimension_semantics=("arbitrary",)),
    )(acc, cnt2, v_tok)


@jax.jit
def _run(x, sims, ln_w, ln_b, q_w, k_w, v_w, indices):
    qk_tok, v_tok = _ln_qkv(x, ln_w.reshape(1, C), ln_b.reshape(1, C),
                            q_w, k_w, v_w)
    gidx = (indices.reshape(B, N)
            + (jnp.arange(B, dtype=jnp.int32) * HW)[:, None]).reshape(BN)
    qk_g, v_g = _sc_gather(qk_tok, v_tok, gidx)
    out_tok = _attention(qk_g, v_g, sims.reshape(BN // _RWS, 1, _RWS))
    acc, cnt2 = _sc_scatter(out_tok, gidx)
    return _merge(acc, cnt2, v_tok)


def kernel(x, sims, mask, ln_w, ln_b, q_w, k_w, v_w, indices, labels,
           num_spixels):
    del mask, labels, num_spixels
    return _run(x, sims, ln_w, ln_b, q_w, k_w, v_w, indices)


# speedup vs baseline: 1.8813x; 1.1705x over previous
import jax
import jax.numpy as jnp
from jax import lax
from jax.experimental import pallas as pl
from jax.experimental.pallas import tpu as pltpu
from jax.experimental.pallas import tpu_sc as plsc

B, C, H, W = 2, 96, 384, 384
QK_DIM = 96
NUM_HEADS = 3
K_SP = 576
TOPK = 64
HEAD_DIM = QK_DIM // NUM_HEADS
SC_SCALE = HEAD_DIM ** (-0.5)
HW = H * W
N = K_SP * TOPK
BN = B * N

NC, NS = 2, 16
NW = NC * NS

_HB1 = 8
_T1 = _HB1 * W


def _k1_body(x_ref, lnw_ref, lnb_ref, qw_ref, kw_ref, vw_ref, qk_ref, v_ref):
    x2 = jnp.reshape(x_ref[0], (C, _T1))
    xt = jnp.transpose(x2, (1, 0))
    mu = jnp.mean(xt, axis=1, keepdims=True)
    var = jnp.mean((xt - mu) ** 2, axis=1, keepdims=True)
    xn = (xt - mu) * lax.rsqrt(var + 1e-6)
    xn = xn * lnw_ref[:] + lnb_ref[:]
    dn = (((1,), (1,)), ((), ()))
    q = lax.dot_general(xn, qw_ref[:], dn, preferred_element_type=jnp.float32)
    k = lax.dot_general(xn, kw_ref[:], dn, preferred_element_type=jnp.float32)
    v = lax.dot_general(xn, vw_ref[:], dn, preferred_element_type=jnp.float32)
    qk_ref[...] = jnp.concatenate([q, k], axis=1)
    v_ref[...] = v


def _ln_qkv(x, ln_w2, ln_b2, q_w, k_w, v_w):
    grid = (B, H // _HB1)
    return pl.pallas_call(
        _k1_body,
        grid=grid,
        in_specs=[
            pl.BlockSpec((1, C, _HB1, W), lambda b, t: (b, 0, t, 0)),
            pl.BlockSpec((1, C), lambda b, t: (0, 0)),
            pl.BlockSpec((1, C), lambda b, t: (0, 0)),
            pl.BlockSpec((C, C), lambda b, t: (0, 0)),
            pl.BlockSpec((C, C), lambda b, t: (0, 0)),
            pl.BlockSpec((C, C), lambda b, t: (0, 0)),
        ],
        out_specs=[
            pl.BlockSpec((_T1, 2 * C), lambda b, t: (b * (H // _HB1) + t, 0)),
            pl.BlockSpec((_T1, C), lambda b, t: (b * (H // _HB1) + t, 0)),
        ],
        out_shape=[
            jax.ShapeDtypeStruct((B * HW, 2 * C), jnp.float32),
            jax.ShapeDtypeStruct((B * HW, C), jnp.float32),
        ],
        compiler_params=pltpu.CompilerParams(
            dimension_semantics=("parallel", "parallel")),
    )(x, ln_w2, ln_b2, q_w, k_w, v_w)


_G_ROWS = BN // NW
_G_CH = 96


def _k2_body(qk_hbm, v_hbm, gidx_hbm, qkg_hbm, vg_hbm, idx_v, qbuf, vbuf, sem):
    wid = lax.axis_index("s") * NC + lax.axis_index("c")
    base = wid * _G_ROWS
    pltpu.sync_copy(gidx_hbm.at[pl.ds(base, _G_ROWS)], idx_v)

    def chunk(i, carry):
        off = i * _G_CH
        pltpu.async_copy(qk_hbm.at[idx_v.at[pl.ds(off, _G_CH)]], qbuf, sem).wait()
        pltpu.sync_copy(qbuf, qkg_hbm.at[pl.ds(base + off, _G_CH)])
        pltpu.async_copy(v_hbm.at[idx_v.at[pl.ds(off, _G_CH)]], vbuf, sem).wait()
        pltpu.sync_copy(vbuf, vg_hbm.at[pl.ds(base + off, _G_CH)])
        return carry

    lax.fori_loop(0, _G_ROWS // _G_CH, chunk, 0)


def _sc_gather(qk_tab, v_tab, gidx):
    mesh = plsc.VectorSubcoreMesh(core_axis_name="c", subcore_axis_name="s")
    f = pl.kernel(
        _k2_body,
        out_type=[
            jax.ShapeDtypeStruct((BN, 2 * C), jnp.float32),
            jax.ShapeDtypeStruct((BN, C), jnp.float32),
        ],
        mesh=mesh,
        scratch_types=[
            pltpu.VMEM((_G_ROWS,), jnp.int32),
            pltpu.VMEM((_G_CH, 2 * C), jnp.float32),
            pltpu.VMEM((_G_CH, C), jnp.float32),
            pltpu.SemaphoreType.DMA,
        ],
        compiler_params=pltpu.CompilerParams(use_tc_tiling_on_sc=False),
    )
    return f(qk_tab, v_tab, gidx)


_GSP = 8
_RWS = _GSP * TOPK


def _k3_body(qk_ref, v_ref, sims_ref, out_ref, asc):
    X = qk_ref[...]
    XV = v_ref[...]
    sT = jnp.transpose(sims_ref[0], (1, 0))

    @pl.when(pl.program_id(0) == 0)
    def _():
        asc[...] = jnp.zeros((_RWS, _RWS), jnp.float32)

    outs = []
    for h in range(NUM_HEADS):
        q = X[:, h * HEAD_DIM:(h + 1) * HEAD_DIM]
        k = X[:, C + h * HEAD_DIM:C + (h + 1) * HEAD_DIM]
        v = XV[:, h * HEAD_DIM:(h + 1) * HEAD_DIM]
        qn = jnp.sum(q * q, axis=1, keepdims=True)
        kn = jnp.sum(k * k, axis=1, keepdims=True)
        dn = (((1,), (1,)), ((), ()))
        QK = lax.dot_general(q, k, dn, preferred_element_type=jnp.float32)
        qkd = jnp.concatenate(
            [QK[s * TOPK:(s + 1) * TOPK, s * TOPK:(s + 1) * TOPK]
             for s in range(_GSP)], axis=0)
        knd = jnp.concatenate(
            [jnp.broadcast_to(
                jnp.transpose(kn[s * TOPK:(s + 1) * TOPK], (1, 0)),
                (TOPK, TOPK)) for s in range(_GSP)], axis=0)
        d2 = qn + knd - 2.0 * qkd
        dist = jnp.sqrt(jnp.maximum(d2, 1e-12))
        a = -SC_SCALE * dist
        a = a - jnp.max(a, axis=1, keepdims=True)
        e = jnp.exp(a)
        attn = e / jnp.sum(e, axis=1, keepdims=True)
        for s in range(_GSP):
            asc[s * TOPK:(s + 1) * TOPK, s * TOPK:(s + 1) * TOPK] = (
                attn[s * TOPK:(s + 1) * TOPK, :])
        vw = v * sT
        dn2 = (((1,), (0,)), ((), ()))
        o = lax.dot_general(asc[...], vw, dn2,
                            preferred_element_type=jnp.float32)
        outs.append(o * sT)
    out_ref[...] = jnp.concatenate(outs, axis=1)


def _attention(qk_g, v_g, sims2):
    grid = (BN // _RWS,)
    return pl.pallas_call(
        _k3_body,
        grid=grid,
        in_specs=[
            pl.BlockSpec((_RWS, 2 * C), lambda i: (i, 0)),
            pl.BlockSpec((_RWS, C), lambda i: (i, 0)),
            pl.BlockSpec((1, 1, _RWS), lambda i: (i, 0, 0)),
        ],
        out_specs=pl.BlockSpec((_RWS, C), lambda i: (i, 0)),
        out_shape=jax.ShapeDtypeStruct((BN, C), jnp.float32),
        scratch_shapes=[pltpu.VMEM((_RWS, _RWS), jnp.float32)],
        compiler_params=pltpu.CompilerParams(
            dimension_semantics=("arbitrary",)),
    )(qk_g, v_g, sims2)


_R = 12288
_NRANGE = HW // _R
_S_TOK = N // NS
_S_CH = 128
_NCH = _S_TOK // _S_CH
_ZROWS = 64
_CROWS = HW // NS // 512


def _k4_body(tok_hbm, gidx_hbm, acc_hbm, cnt_hbm,
             idxa_v, idxb_v, cidx, cdst, cidx_c, cdst_c, bufa, zbuf, cntv,
             sema, semz, acc_sp):
    c = lax.axis_index("c")
    s = lax.axis_index("s")
    base_tok = c * N + s * _S_TOK
    ones16 = jnp.full((16,), 1.0, jnp.float32)

    with jax.named_scope("k4_init"):
        def zfill(i, carry):
            def zf2(j, carry2):
                zbuf[i, pl.ds(j * 16, 16)] = jnp.zeros((16,), jnp.float32)
                return carry2
            lax.fori_loop(0, C // 16, zf2, 0)
            return carry
        lax.fori_loop(0, _ZROWS, zfill, 0)

        def cfill(i, carry):
            cntv[pl.ds(i * 16, 16)] = jnp.zeros((16,), jnp.float32)
            return carry
        lax.fori_loop(0, (HW // NS) // 16, cfill, 0)

    with jax.named_scope("k4_cnt"):
        pix0 = c * HW + s * (HW // NS)
        pltpu.async_copy(gidx_hbm.at[pl.ds(c * N, _S_TOK)], idxa_v, sema)

        def scan_tile(tt, carry):
            pltpu.make_async_copy(gidx_hbm.at[pl.ds(0, _S_TOK)],
                                  idxa_v, sema).wait()
            def mv(i, carry2):
                idxb_v[pl.ds(i * 16, 16)] = idxa_v[pl.ds(i * 16, 16)]
                return carry2
            lax.fori_loop(0, _S_TOK // 16, mv, 0)

            @pl.when(tt + 1 < NS)
            def _():
                pltpu.async_copy(
                    gidx_hbm.at[pl.ds(c * N + (tt + 1) * _S_TOK, _S_TOK)],
                    idxa_v, sema)

            def scan(i, carry2):
                vv = idxb_v[pl.ds(i * 16, 16)] - pix0
                msk = (vv >= 0) & (vv < HW // NS)
                vv = jnp.where(msk, vv, 0)
                plsc.addupdate_scatter(cntv, [vv], ones16, mask=msk)
                return carry2
            lax.fori_loop(0, _S_TOK // 16, scan, 0)
            return carry
        lax.fori_loop(0, NS, scan_tile, 0)
        pltpu.sync_copy(cntv, cnt_hbm.at[0, pl.ds(pix0, HW // NS)])

    pltpu.sync_copy(gidx_hbm.at[pl.ds(base_tok, _S_TOK)], idxa_v)
    iota16 = lax.iota(jnp.int32, 16)

    def one_range(rr, carry):
        r0g = c * HW + rr * _R
        with jax.named_scope("k4_zero"):
            def za(i, carry2):
                pltpu.async_copy(
                    zbuf,
                    acc_sp.at[pl.ds(s * (_R // NS) + i * _ZROWS, _ZROWS)],
                    semz)
                return carry2
            lax.fori_loop(0, _R // NS // _ZROWS, za, 0)

        with jax.named_scope("k4_compact"):
            def cp(i, off):
                vv = idxa_v[pl.ds(i * 16, 16)] - r0g
                msk = (vv >= 0) & (vv < _R)
                rows = base_tok + i * 16 + iota16
                plsc.store_compressed(cidx.at[pl.ds(off, 16)], rows, mask=msk)
                plsc.store_compressed(cdst.at[pl.ds(off, 16)], vv, mask=msk)
                pc = plsc.all_reduce_population_count(msk)
                return off + pc[0]
            nc = lax.fori_loop(0, _S_TOK // 16, cp, 0)
            def pad(k, carry2):
                cidx[pl.ds(nc + k * 16, 16)] = jnp.full((16,), base_tok,
                                                        jnp.int32)
                cdst[pl.ds(nc + k * 16, 16)] = jnp.full((16,), _R, jnp.int32)
                return carry2
            lax.fori_loop(0, _S_CH // 16, pad, 0)

        with jax.named_scope("k4_zdrain"):
            def zd(i, carry2):
                pltpu.make_async_copy(
                    zbuf, acc_sp.at[pl.ds(s * (_R // NS), _ZROWS)],
                    semz).wait()
                return carry2
            lax.fori_loop(0, _R // NS // _ZROWS, zd, 0)
        plsc.subcore_barrier()

        with jax.named_scope("k4_chunks"):
            nch = lax.shift_right_logical(nc + (_S_CH - 1), 7)

            def chunk(j, carry2):
                def mv(k, carry3):
                    cidx_c[pl.ds(k * 16, 16)] = (
                        cidx[pl.ds(j * _S_CH + k * 16, 16)])
                    cdst_c[pl.ds(k * 16, 16)] = (
                        cdst[pl.ds(j * _S_CH + k * 16, 16)])
                    return carry3
                lax.fori_loop(0, _S_CH // 16, mv, 0)
                pltpu.async_copy(tok_hbm.at[cidx_c], bufa, sema).wait()
                pltpu.sync_copy(bufa, acc_sp.at[cdst_c], add=True)
                return carry2
            lax.fori_loop(0, nch, chunk, 0)
        plsc.subcore_barrier()

        with jax.named_scope("k4_out"):
            pltpu.sync_copy(acc_sp.at[pl.ds(s * (_R // NS), _R // NS)],
                            acc_hbm.at[pl.ds(r0g + s * (_R // NS), _R // NS)])
        plsc.subcore_barrier()
        return carry

    lax.fori_loop(0, _NRANGE, one_range, 0)


def _sc_scatter(out_tok, gidx):
    mesh = plsc.VectorSubcoreMesh(core_axis_name="c", subcore_axis_name="s")
    f = pl.kernel(
        _k4_body,
        out_type=[
            jax.ShapeDtypeStruct((B * HW, C), jnp.float32),
            jax.ShapeDtypeStruct((1, B * HW), jnp.float32),
        ],
        mesh=mesh,
        scratch_types=[
            pltpu.VMEM((_S_TOK,), jnp.int32),
            pltpu.VMEM((_S_TOK,), jnp.int32),
            pltpu.VMEM((_S_TOK + _S_CH,), jnp.int32),
            pltpu.VMEM((_S_TOK + _S_CH,), jnp.int32),
            pltpu.VMEM((_S_CH,), jnp.int32),
            pltpu.VMEM((_S_CH,), jnp.int32),
            pltpu.VMEM((_S_CH, C), jnp.float32),
            pltpu.VMEM((_ZROWS, C), jnp.float32),
            pltpu.VMEM((HW // NS,), jnp.float32),
            pltpu.SemaphoreType.DMA,
            pltpu.SemaphoreType.DMA,
            pltpu.VMEM_SHARED((_R + 16, C), jnp.float32),
        ],
        compiler_params=pltpu.CompilerParams(use_tc_tiling_on_sc=False,
                                             needs_layout_passes=False),
    )
    return f(out_tok, gidx)


_T5 = 12288
_HB5 = _T5 // W


def _k5_body(acc_ref, cnt_ref, v_ref, out_ref):
    a = acc_ref[...]
    ct = jnp.transpose(cnt_ref[...], (1, 0))
    v = v_ref[...]
    mean = a / jnp.maximum(ct, 1.0)
    res = jnp.where(ct > 1e-5, mean, v)
    rT = jnp.transpose(res, (1, 0))
    for hb in range(_HB5):
        out_ref[0, :, hb, :] = rT[:, hb * W:(hb + 1) * W]


def _merge(acc, cnt2, v_tok):
    grid = (B * HW // _T5,)
    nh = H // _HB5
    return pl.pallas_call(
        _k5_body,
        grid=grid,
        in_specs=[
            pl.BlockSpec((_T5, C), lambda t: (t, 0)),
            pl.BlockSpec((1, _T5), lambda t: (0, t)),
            pl.BlockSpec((_T5, C), lambda t: (t, 0)),
        ],
        out_specs=pl.BlockSpec((1, C, _HB5, W), lambda t: (t // nh, 0, t % nh, 0)),
        out_shape=jax.ShapeDtypeStruct((B, C, H, W), jnp.float32),
        compiler_params=pltpu.CompilerParams(
            dimension_semantics=("arbitrary",)),
    )(acc, cnt2, v_tok)


@jax.jit
def _run(x, sims, ln_w, ln_b, q_w, k_w, v_w, indices):
    qk_tok, v_tok = _ln_qkv(x, ln_w.reshape(1, C), ln_b.reshape(1, C),
                            q_w, k_w, v_w)
    gidx = (indices.reshape(B, N)
            + (jnp.arange(B, dtype=jnp.int32) * HW)[:, None]).reshape(BN)
    qk_g, v_g = _sc_gather(qk_tok, v_tok, gidx)
    out_tok = _attention(qk_g, v_g, sims.reshape(BN // _RWS, 1, _RWS))
    acc, cnt2 = _sc_scatter(out_tok, gidx)
    return _merge(acc, cnt2, v_tok)


def kernel(x, sims, mask, ln_w, ln_b, q_w, k_w, v_w, indices, labels,
           num_spixels):
    del mask, labels, num_spixels
    return _run(x, sims, ln_w, ln_b, q_w, k_w, v_w, indices)


# speedup vs baseline: 2.6705x; 1.4195x over previous
import jax
import jax.numpy as jnp
from jax import lax
from jax.experimental import pallas as pl
from jax.experimental.pallas import tpu as pltpu
from jax.experimental.pallas import tpu_sc as plsc

B, C, H, W = 2, 96, 384, 384
QK_DIM = 96
NUM_HEADS = 3
K_SP = 576
TOPK = 64
HEAD_DIM = QK_DIM // NUM_HEADS
SC_SCALE = HEAD_DIM ** (-0.5)
HW = H * W
N = K_SP * TOPK
BN = B * N

NC, NS = 2, 16
NW = NC * NS

_HB1 = 8
_T1 = _HB1 * W


def _k1_body(x_ref, lnw_ref, lnb_ref, qw_ref, kw_ref, vw_ref, qk_ref, v_ref):
    x2 = jnp.reshape(x_ref[0], (C, _T1))
    xt = jnp.transpose(x2, (1, 0))
    mu = jnp.mean(xt, axis=1, keepdims=True)
    var = jnp.mean((xt - mu) ** 2, axis=1, keepdims=True)
    xn = (xt - mu) * lax.rsqrt(var + 1e-6)
    xn = xn * lnw_ref[:] + lnb_ref[:]
    dn = (((1,), (1,)), ((), ()))
    q = lax.dot_general(xn, qw_ref[:], dn, preferred_element_type=jnp.float32)
    k = lax.dot_general(xn, kw_ref[:], dn, preferred_element_type=jnp.float32)
    v = lax.dot_general(xn, vw_ref[:], dn, preferred_element_type=jnp.float32)
    z32 = jnp.zeros((_T1, 32), jnp.float32)
    qk_ref[...] = jnp.concatenate([q, z32, k, z32], axis=1)
    v_ref[...] = jnp.concatenate([v, z32], axis=1)


def _ln_qkv(x, ln_w2, ln_b2, q_w, k_w, v_w):
    grid = (B, H // _HB1)
    return pl.pallas_call(
        _k1_body,
        grid=grid,
        in_specs=[
            pl.BlockSpec((1, C, _HB1, W), lambda b, t: (b, 0, t, 0)),
            pl.BlockSpec((1, C), lambda b, t: (0, 0)),
            pl.BlockSpec((1, C), lambda b, t: (0, 0)),
            pl.BlockSpec((C, C), lambda b, t: (0, 0)),
            pl.BlockSpec((C, C), lambda b, t: (0, 0)),
            pl.BlockSpec((C, C), lambda b, t: (0, 0)),
        ],
        out_specs=[
            pl.BlockSpec((_T1, 256), lambda b, t: (b * (H // _HB1) + t, 0)),
            pl.BlockSpec((_T1, 128), lambda b, t: (b * (H // _HB1) + t, 0)),
        ],
        out_shape=[
            jax.ShapeDtypeStruct((B * HW, 256), jnp.float32),
            jax.ShapeDtypeStruct((B * HW, 128), jnp.float32),
        ],
        compiler_params=pltpu.CompilerParams(
            dimension_semantics=("parallel", "parallel")),
    )(x, ln_w2, ln_b2, q_w, k_w, v_w)


_G_ROWS = BN // NW
_G_CH = 96


def _k2_body(qk_hbm, v_hbm, gidx_hbm, qkg_hbm, vg_hbm, idx_v, qbuf, vbuf, sem):
    wid = lax.axis_index("s") * NC + lax.axis_index("c")
    base = wid * _G_ROWS
    pltpu.sync_copy(gidx_hbm.at[pl.ds(base, _G_ROWS)], idx_v)

    def chunk(i, carry):
        off = i * _G_CH
        pltpu.async_copy(qk_hbm.at[idx_v.at[pl.ds(off, _G_CH)]], qbuf, sem).wait()
        pltpu.sync_copy(qbuf, qkg_hbm.at[pl.ds(base + off, _G_CH)])
        pltpu.async_copy(v_hbm.at[idx_v.at[pl.ds(off, _G_CH)]], vbuf, sem).wait()
        pltpu.sync_copy(vbuf, vg_hbm.at[pl.ds(base + off, _G_CH)])
        return carry

    lax.fori_loop(0, _G_ROWS // _G_CH, chunk, 0)


def _sc_gather(qk_tab, v_tab, gidx):
    mesh = plsc.VectorSubcoreMesh(core_axis_name="c", subcore_axis_name="s")
    f = pl.kernel(
        _k2_body,
        out_type=[
            jax.ShapeDtypeStruct((BN, 256), jnp.float32),
            jax.ShapeDtypeStruct((BN, 128), jnp.float32),
        ],
        mesh=mesh,
        scratch_types=[
            pltpu.VMEM((_G_ROWS,), jnp.int32),
            pltpu.VMEM((_G_CH, 256), jnp.float32),
            pltpu.VMEM((_G_CH, 128), jnp.float32),
            pltpu.SemaphoreType.DMA,
        ],
        compiler_params=pltpu.CompilerParams(use_tc_tiling_on_sc=True),
    )
    return f(qk_tab, v_tab, gidx)


_GSP = 8
_RWS = _GSP * TOPK


def _k3_body(qk_ref, v_ref, sims_ref, out_ref):
    X = qk_ref[...]
    XV = v_ref[...]
    sT = jnp.transpose(sims_ref[0], (1, 0))
    rb = lax.shift_right_logical(
        lax.broadcasted_iota(jnp.int32, (_RWS, _RWS), 0), 6)
    cb = lax.shift_right_logical(
        lax.broadcasted_iota(jnp.int32, (_RWS, _RWS), 1), 6)
    pen = jnp.where(rb == cb, 0.0, -1e30)

    outs = []
    for h in range(NUM_HEADS):
        q = X[:, h * HEAD_DIM:(h + 1) * HEAD_DIM]
        k = X[:, 128 + h * HEAD_DIM:128 + (h + 1) * HEAD_DIM]
        v = XV[:, h * HEAD_DIM:(h + 1) * HEAD_DIM]
        qn = jnp.sum(q * q, axis=1, keepdims=True)
        kn = jnp.sum(k * k, axis=1, keepdims=True)
        dn = (((1,), (1,)), ((), ()))
        QK = lax.dot_general(q, k, dn, preferred_element_type=jnp.float32)
        d2 = qn + jnp.transpose(kn, (1, 0)) - 2.0 * QK
        dist = jnp.sqrt(jnp.maximum(d2, 1e-12))
        a = -SC_SCALE * dist + pen
        a = a - jnp.max(a, axis=1, keepdims=True)
        e = jnp.exp(a)
        ssum = jnp.sum(e, axis=1, keepdims=True)
        vw = v * sT
        dn2 = (((1,), (0,)), ((), ()))
        o = lax.dot_general(e, vw, dn2, preferred_element_type=jnp.float32)
        outs.append(o * (sT / ssum))
    ones1 = jnp.ones((_RWS, 1), jnp.float32)
    z31 = jnp.zeros((_RWS, 31), jnp.float32)
    out_ref[...] = jnp.concatenate(outs + [ones1, z31], axis=1)


def _attention(qk_g, v_g, sims2):
    grid = (BN // _RWS,)
    return pl.pallas_call(
        _k3_body,
        grid=grid,
        in_specs=[
            pl.BlockSpec((_RWS, 256), lambda i: (i, 0)),
            pl.BlockSpec((_RWS, 128), lambda i: (i, 0)),
            pl.BlockSpec((1, 1, _RWS), lambda i: (i, 0, 0)),
        ],
        out_specs=pl.BlockSpec((_RWS, 128), lambda i: (i, 0)),
        out_shape=jax.ShapeDtypeStruct((BN, 128), jnp.float32),
        compiler_params=pltpu.CompilerParams(
            dimension_semantics=("arbitrary",)),
    )(qk_g, v_g, sims2)


_R = 9216
_NRANGE = HW // _R
_S_TOK = N // NS
_S_CH = 128
_ZROWS = 64


def _k4_body(tok_hbm, gidx_hbm, acc_hbm,
             idxa_v, cidx, cdst, cidx_c, cdst_c, bufa, zbuf,
             sema, semz, acc_sp):
    c = lax.axis_index("c")
    s = lax.axis_index("s")
    base_tok = c * N + s * _S_TOK

    with jax.named_scope("k4_init"):
        def zfill(i, carry):
            def zf2(j, carry2):
                zbuf[i, pl.ds(j * 16, 16)] = jnp.zeros((16,), jnp.float32)
                return carry2
            lax.fori_loop(0, 128 // 16, zf2, 0)
            return carry
        lax.fori_loop(0, _ZROWS, zfill, 0)

    pltpu.sync_copy(gidx_hbm.at[pl.ds(base_tok, _S_TOK)], idxa_v)
    iota16 = lax.iota(jnp.int32, 16)

    def one_range(rr, carry):
        r0g = c * HW + rr * _R
        with jax.named_scope("k4_zero"):
            def za(i, carry2):
                pltpu.async_copy(
                    zbuf,
                    acc_sp.at[pl.ds(s * (_R // NS) + i * _ZROWS, _ZROWS)],
                    semz)
                return carry2
            lax.fori_loop(0, _R // NS // _ZROWS, za, 0)

        with jax.named_scope("k4_compact"):
            def cp(i, off):
                vv = idxa_v[pl.ds(i * 16, 16)] - r0g
                msk = (vv >= 0) & (vv < _R)
                rows = base_tok + i * 16 + iota16
                plsc.store_compressed(cidx.at[pl.ds(off, 16)], rows, mask=msk)
                plsc.store_compressed(cdst.at[pl.ds(off, 16)], vv, mask=msk)
                pc = plsc.all_reduce_population_count(msk)
                return off + pc[0]
            nc = lax.fori_loop(0, _S_TOK // 16, cp, 0)
            def pad(k, carry2):
                cidx[pl.ds(nc + k * 16, 16)] = jnp.full((16,), base_tok,
                                                        jnp.int32)
                cdst[pl.ds(nc + k * 16, 16)] = jnp.full((16,), _R, jnp.int32)
                return carry2
            lax.fori_loop(0, _S_CH // 16, pad, 0)

        with jax.named_scope("k4_zdrain"):
            def zd(i, carry2):
                pltpu.make_async_copy(
                    zbuf, acc_sp.at[pl.ds(s * (_R // NS), _ZROWS)],
                    semz).wait()
                return carry2
            lax.fori_loop(0, _R // NS // _ZROWS, zd, 0)
        plsc.subcore_barrier()

        with jax.named_scope("k4_chunks"):
            nch = lax.shift_right_logical(nc + (_S_CH - 1), 7)

            def chunk(j, carry2):
                def mv(k, carry3):
                    cidx_c[pl.ds(k * 16, 16)] = (
                        cidx[pl.ds(j * _S_CH + k * 16, 16)])
                    cdst_c[pl.ds(k * 16, 16)] = (
                        cdst[pl.ds(j * _S_CH + k * 16, 16)])
                    return carry3
                lax.fori_loop(0, _S_CH // 16, mv, 0)
                pltpu.async_copy(tok_hbm.at[cidx_c], bufa, sema).wait()
                pltpu.sync_copy(bufa, acc_sp.at[cdst_c], add=True)
                return carry2
            lax.fori_loop(0, nch, chunk, 0)
        plsc.subcore_barrier()

        with jax.named_scope("k4_out"):
            pltpu.sync_copy(acc_sp.at[pl.ds(s * (_R // NS), _R // NS)],
                            acc_hbm.at[pl.ds(r0g + s * (_R // NS), _R // NS)])
        plsc.subcore_barrier()
        return carry

    lax.fori_loop(0, _NRANGE, one_range, 0)


def _sc_scatter(out_tok, gidx):
    mesh = plsc.VectorSubcoreMesh(core_axis_name="c", subcore_axis_name="s")
    f = pl.kernel(
        _k4_body,
        out_type=jax.ShapeDtypeStruct((B * HW, 128), jnp.float32),
        mesh=mesh,
        scratch_types=[
            pltpu.VMEM((_S_TOK,), jnp.int32),
            pltpu.VMEM((_S_TOK + _S_CH,), jnp.int32),
            pltpu.VMEM((_S_TOK + _S_CH,), jnp.int32),
            pltpu.VMEM((_S_CH,), jnp.int32),
            pltpu.VMEM((_S_CH,), jnp.int32),
            pltpu.VMEM((_S_CH, 128), jnp.float32),
            pltpu.VMEM((_ZROWS, 128), jnp.float32),
            pltpu.SemaphoreType.DMA,
            pltpu.SemaphoreType.DMA,
            pltpu.VMEM_SHARED((_R + 16, 128), jnp.float32),
        ],
        compiler_params=pltpu.CompilerParams(use_tc_tiling_on_sc=True,
                                             needs_layout_passes=False),
    )
    return f(out_tok, gidx)


_T5 = 12288
_HB5 = _T5 // W


def _k5_body(acc_ref, v_ref, out_ref):
    a = acc_ref[...]
    ct = a[:, C:C + 1]
    v = v_ref[:, 0:C]
    mean = a[:, 0:C] / jnp.maximum(ct, 1.0)
    res = jnp.where(ct > 1e-5, mean, v)
    rT = jnp.transpose(res, (1, 0))
    for hb in range(_HB5):
        out_ref[0, :, hb, :] = rT[:, hb * W:(hb + 1) * W]


def _merge(acc, v_tok):
    grid = (B * HW // _T5,)
    nh = H // _HB5
    return pl.pallas_call(
        _k5_body,
        grid=grid,
        in_specs=[
            pl.BlockSpec((_T5, 128), lambda t: (t, 0)),
            pl.BlockSpec((_T5, 128), lambda t: (t, 0)),
        ],
        out_specs=pl.BlockSpec((1, C, _HB5, W), lambda t: (t // nh, 0, t % nh, 0)),
        out_shape=jax.ShapeDtypeStruct((B, C, H, W), jnp.float32),
        compiler_params=pltpu.CompilerParams(
            dimension_semantics=("arbitrary",)),
    )(acc, v_tok)


@jax.jit
def _run(x, sims, ln_w, ln_b, q_w, k_w, v_w, indices):
    qk_tok, v_tok = _ln_qkv(x, ln_w.reshape(1, C), ln_b.reshape(1, C),
                            q_w, k_w, v_w)
    gidx = (indices.reshape(B, N)
            + (jnp.arange(B, dtype=jnp.int32) * HW)[:, None]).reshape(BN)
    qk_g, v_g = _sc_gather(qk_tok, v_tok, gidx)
    out_tok = _attention(qk_g, v_g, sims.reshape(BN // _RWS, 1, _RWS))
    acc = _sc_scatter(out_tok, gidx)
    return _merge(acc, v_tok)


def kernel(x, sims, mask, ln_w, ln_b, q_w, k_w, v_w, indices, labels,
           num_spixels):
    del mask, labels, num_spixels
    return _run(x, sims, ln_w, ln_b, q_w, k_w, v_w, indices)


# speedup vs baseline: 2.9356x; 1.0992x over previous
import jax
import jax.numpy as jnp
from jax import lax
from jax.experimental import pallas as pl
from jax.experimental.pallas import tpu as pltpu
from jax.experimental.pallas import tpu_sc as plsc

B, C, H, W = 2, 96, 384, 384
QK_DIM = 96
NUM_HEADS = 3
K_SP = 576
TOPK = 64
HEAD_DIM = QK_DIM // NUM_HEADS
SC_SCALE = HEAD_DIM ** (-0.5)
HW = H * W
N = K_SP * TOPK
BN = B * N

NC, NS = 2, 16
NW = NC * NS

_HB1 = 8
_T1 = _HB1 * W


def _k1_body(x_ref, lnw_ref, lnb_ref, qw_ref, kw_ref, vw_ref, qk_ref, v_ref):
    x2 = jnp.reshape(x_ref[0], (C, _T1))
    xt = jnp.transpose(x2, (1, 0))
    mu = jnp.mean(xt, axis=1, keepdims=True)
    var = jnp.mean((xt - mu) ** 2, axis=1, keepdims=True)
    xn = (xt - mu) * lax.rsqrt(var + 1e-6)
    xn = xn * lnw_ref[:] + lnb_ref[:]
    dn = (((1,), (1,)), ((), ()))
    q = lax.dot_general(xn, qw_ref[:], dn, preferred_element_type=jnp.float32)
    k = lax.dot_general(xn, kw_ref[:], dn, preferred_element_type=jnp.float32)
    v = lax.dot_general(xn, vw_ref[:], dn, preferred_element_type=jnp.float32)
    z32 = jnp.zeros((_T1, 32), jnp.float32)
    qk_ref[...] = jnp.concatenate([q, z32, k, z32], axis=1)
    v_ref[...] = jnp.concatenate([v, z32], axis=1)


def _ln_qkv(x, ln_w2, ln_b2, q_w, k_w, v_w):
    grid = (B, H // _HB1)
    return pl.pallas_call(
        _k1_body,
        grid=grid,
        in_specs=[
            pl.BlockSpec((1, C, _HB1, W), lambda b, t: (b, 0, t, 0)),
            pl.BlockSpec((1, C), lambda b, t: (0, 0)),
            pl.BlockSpec((1, C), lambda b, t: (0, 0)),
            pl.BlockSpec((C, C), lambda b, t: (0, 0)),
            pl.BlockSpec((C, C), lambda b, t: (0, 0)),
            pl.BlockSpec((C, C), lambda b, t: (0, 0)),
        ],
        out_specs=[
            pl.BlockSpec((_T1, 256), lambda b, t: (b * (H // _HB1) + t, 0)),
            pl.BlockSpec((_T1, 128), lambda b, t: (b * (H // _HB1) + t, 0)),
        ],
        out_shape=[
            jax.ShapeDtypeStruct((B * HW, 256), jnp.float32),
            jax.ShapeDtypeStruct((B * HW, 128), jnp.float32),
        ],
        compiler_params=pltpu.CompilerParams(
            dimension_semantics=("parallel", "parallel")),
    )(x, ln_w2, ln_b2, q_w, k_w, v_w)


_G_ROWS = BN // NW
_G_CH = 96


def _k2_body(qk_hbm, v_hbm, gidx_hbm, qkg_hbm, vg_hbm, idx_v, qbuf, vbuf, sem):
    wid = lax.axis_index("s") * NC + lax.axis_index("c")
    base = wid * _G_ROWS
    pltpu.sync_copy(gidx_hbm.at[pl.ds(base, _G_ROWS)], idx_v)

    def chunk(i, carry):
        off = i * _G_CH
        pltpu.async_copy(qk_hbm.at[idx_v.at[pl.ds(off, _G_CH)]], qbuf, sem).wait()
        pltpu.sync_copy(qbuf, qkg_hbm.at[pl.ds(base + off, _G_CH)])
        pltpu.async_copy(v_hbm.at[idx_v.at[pl.ds(off, _G_CH)]], vbuf, sem).wait()
        pltpu.sync_copy(vbuf, vg_hbm.at[pl.ds(base + off, _G_CH)])
        return carry

    lax.fori_loop(0, _G_ROWS // _G_CH, chunk, 0)


def _sc_gather(qk_tab, v_tab, gidx):
    mesh = plsc.VectorSubcoreMesh(core_axis_name="c", subcore_axis_name="s")
    f = pl.kernel(
        _k2_body,
        out_type=[
            jax.ShapeDtypeStruct((BN, 256), jnp.float32),
            jax.ShapeDtypeStruct((BN, 128), jnp.float32),
        ],
        mesh=mesh,
        scratch_types=[
            pltpu.VMEM((_G_ROWS,), jnp.int32),
            pltpu.VMEM((_G_CH, 256), jnp.float32),
            pltpu.VMEM((_G_CH, 128), jnp.float32),
            pltpu.SemaphoreType.DMA,
        ],
        compiler_params=pltpu.CompilerParams(use_tc_tiling_on_sc=True),
    )
    return f(qk_tab, v_tab, gidx)


_GSP = 8
_RWS = _GSP * TOPK


def _k3_body(qk_ref, v_ref, sims_ref, out_ref, pen_sc):
    X = qk_ref[...]
    XV = v_ref[...]
    sT = jnp.transpose(sims_ref[0], (1, 0))
    ones1 = jnp.ones((_RWS, 1), jnp.float32)

    @pl.when(pl.program_id(0) == 0)
    def _():
        rb = lax.shift_right_logical(
            lax.broadcasted_iota(jnp.int32, (_RWS, _RWS), 0), 6)
        cb = lax.shift_right_logical(
            lax.broadcasted_iota(jnp.int32, (_RWS, _RWS), 1), 6)
        pen_sc[...] = jnp.where(rb == cb, 0.0, -1e30)

    pen = pen_sc[...]
    outs = []
    dn = (((1,), (1,)), ((), ()))
    for h in range(NUM_HEADS):
        q = X[:, h * HEAD_DIM:(h + 1) * HEAD_DIM]
        k = X[:, 128 + h * HEAD_DIM:128 + (h + 1) * HEAD_DIM]
        v = XV[:, h * HEAD_DIM:(h + 1) * HEAD_DIM]
        qn = jnp.sum(q * q, axis=1, keepdims=True)
        kn = jnp.sum(k * k, axis=1, keepdims=True)
        qa = jnp.concatenate([-2.0 * q, ones1], axis=1)
        ka = jnp.concatenate([k, kn], axis=1)
        d2k = lax.dot_general(qa, ka, dn,
                              preferred_element_type=jnp.float32)
        d2 = d2k + qn
        dist = jnp.sqrt(jnp.maximum(d2, 1e-12))
        e = jnp.exp(pen - SC_SCALE * dist)
        vwa = jnp.concatenate([v * sT, ones1], axis=1)
        dn2 = (((1,), (0,)), ((), ()))
        os = lax.dot_general(e, vwa, dn2,
                             preferred_element_type=jnp.float32)
        outs.append(os[:, 0:HEAD_DIM] * (sT / os[:, HEAD_DIM:HEAD_DIM + 1]))
    z31 = jnp.zeros((_RWS, 31), jnp.float32)
    out_ref[...] = jnp.concatenate(outs + [ones1, z31], axis=1)


def _attention(qk_g, v_g, sims2):
    grid = (BN // _RWS,)
    return pl.pallas_call(
        _k3_body,
        grid=grid,
        in_specs=[
            pl.BlockSpec((_RWS, 256), lambda i: (i, 0)),
            pl.BlockSpec((_RWS, 128), lambda i: (i, 0)),
            pl.BlockSpec((1, 1, _RWS), lambda i: (i, 0, 0)),
        ],
        out_specs=pl.BlockSpec((_RWS, 128), lambda i: (i, 0)),
        out_shape=jax.ShapeDtypeStruct((BN, 128), jnp.float32),
        scratch_shapes=[pltpu.VMEM((_RWS, _RWS), jnp.float32)],
        compiler_params=pltpu.CompilerParams(
            dimension_semantics=("arbitrary",)),
    )(qk_g, v_g, sims2)


_R = 12288
_NRANGE = HW // _R
_S_TOK = N // NS
_S_CH = 128
_ZROWS = 32


def _k4_body(tok_hbm, gidx_hbm, acc_hbm,
             idxa_v, cidx, cdst, cidx_c, cdst_c, bufa, zbuf,
             sema, semz, acc_sp):
    c = lax.axis_index("c")
    s = lax.axis_index("s")
    base_tok = c * N + s * _S_TOK

    with jax.named_scope("k4_init"):
        def zfill(i, carry):
            def zf2(j, carry2):
                zbuf[i, pl.ds(j * 16, 16)] = jnp.zeros((16,), jnp.float32)
                return carry2
            lax.fori_loop(0, 128 // 16, zf2, 0)
            return carry
        lax.fori_loop(0, _ZROWS, zfill, 0)

    pltpu.sync_copy(gidx_hbm.at[pl.ds(base_tok, _S_TOK)], idxa_v)
    iota16 = lax.iota(jnp.int32, 16)

    def one_range(rr, carry):
        r0g = c * HW + rr * _R
        with jax.named_scope("k4_zero"):
            def za(i, carry2):
                pltpu.async_copy(
                    zbuf,
                    acc_sp.at[pl.ds(s * (_R // NS) + i * _ZROWS, _ZROWS)],
                    semz)
                return carry2
            lax.fori_loop(0, _R // NS // _ZROWS, za, 0)

        with jax.named_scope("k4_compact"):
            def cp(i, off):
                vv = idxa_v[pl.ds(i * 16, 16)] - r0g
                msk = (vv >= 0) & (vv < _R)
                rows = base_tok + i * 16 + iota16
                plsc.store_compressed(cidx.at[pl.ds(off, 16)], rows, mask=msk)
                plsc.store_compressed(cdst.at[pl.ds(off, 16)], vv, mask=msk)
                pc = plsc.all_reduce_population_count(msk)
                return off + pc[0]
            nc = lax.fori_loop(0, _S_TOK // 16, cp, 0)
            def pad(k, carry2):
                cidx[pl.ds(nc + k * 16, 16)] = jnp.full((16,), base_tok,
                                                        jnp.int32)
                cdst[pl.ds(nc + k * 16, 16)] = jnp.full((16,), _R, jnp.int32)
                return carry2
            lax.fori_loop(0, _S_CH // 16, pad, 0)

        with jax.named_scope("k4_zdrain"):
            def zd(i, carry2):
                pltpu.make_async_copy(
                    zbuf, acc_sp.at[pl.ds(s * (_R // NS), _ZROWS)],
                    semz).wait()
                return carry2
            lax.fori_loop(0, _R // NS // _ZROWS, zd, 0)
        plsc.subcore_barrier()

        with jax.named_scope("k4_chunks"):
            nch = lax.shift_right_logical(nc + (_S_CH - 1), 7)

            def chunk(j, carry2):
                def mv(k, carry3):
                    cidx_c[pl.ds(k * 16, 16)] = (
                        cidx[pl.ds(j * _S_CH + k * 16, 16)])
                    cdst_c[pl.ds(k * 16, 16)] = (
                        cdst[pl.ds(j * _S_CH + k * 16, 16)])
                    return carry3
                lax.fori_loop(0, _S_CH // 16, mv, 0)
                pltpu.async_copy(tok_hbm.at[cidx_c], bufa, sema).wait()
                pltpu.sync_copy(bufa, acc_sp.at[cdst_c], add=True)
                return carry2
            lax.fori_loop(0, nch, chunk, 0)
        plsc.subcore_barrier()

        with jax.named_scope("k4_out"):
            pltpu.sync_copy(acc_sp.at[pl.ds(s * (_R // NS), _R // NS)],
                            acc_hbm.at[pl.ds(r0g + s * (_R // NS), _R // NS)])
        plsc.subcore_barrier()
        return carry

    lax.fori_loop(0, _NRANGE, one_range, 0)


def _sc_scatter(out_tok, gidx):
    mesh = plsc.VectorSubcoreMesh(core_axis_name="c", subcore_axis_name="s")
    f = pl.kernel(
        _k4_body,
        out_type=jax.ShapeDtypeStruct((B * HW, 128), jnp.float32),
        mesh=mesh,
        scratch_types=[
            pltpu.VMEM((_S_TOK,), jnp.int32),
            pltpu.VMEM((_S_TOK + _S_CH,), jnp.int32),
            pltpu.VMEM((_S_TOK + _S_CH,), jnp.int32),
            pltpu.VMEM((_S_CH,), jnp.int32),
            pltpu.VMEM((_S_CH,), jnp.int32),
            pltpu.VMEM((_S_CH, 128), jnp.float32),
            pltpu.VMEM((_ZROWS, 128), jnp.float32),
            pltpu.SemaphoreType.DMA,
            pltpu.SemaphoreType.DMA,
            pltpu.VMEM_SHARED((_R + 16, 128), jnp.float32),
        ],
        compiler_params=pltpu.CompilerParams(use_tc_tiling_on_sc=True,
                                             needs_layout_passes=False),
    )
    return f(out_tok, gidx)


_T5 = 12288
_HB5 = _T5 // W


def _k5_body(acc_ref, v_ref, out_ref):
    a = acc_ref[...]
    ct = a[:, C:C + 1]
    v = v_ref[:, 0:C]
    mean = a[:, 0:C] / jnp.maximum(ct, 1.0)
    res = jnp.where(ct > 1e-5, mean, v)
    rT = jnp.transpose(res, (1, 0))
    for hb in range(_HB5):
        out_ref[0, :, hb, :] = rT[:, hb * W:(hb + 1) * W]


def _merge(acc, v_tok):
    grid = (B * HW // _T5,)
    nh = H // _HB5
    return pl.pallas_call(
        _k5_body,
        grid=grid,
        in_specs=[
            pl.BlockSpec((_T5, 128), lambda t: (t, 0)),
            pl.BlockSpec((_T5, 128), lambda t: (t, 0)),
        ],
        out_specs=pl.BlockSpec((1, C, _HB5, W), lambda t: (t // nh, 0, t % nh, 0)),
        out_shape=jax.ShapeDtypeStruct((B, C, H, W), jnp.float32),
        compiler_params=pltpu.CompilerParams(
            dimension_semantics=("arbitrary",)),
    )(acc, v_tok)


@jax.jit
def _run(x, sims, ln_w, ln_b, q_w, k_w, v_w, indices):
    qk_tok, v_tok = _ln_qkv(x, ln_w.reshape(1, C), ln_b.reshape(1, C),
                            q_w, k_w, v_w)
    gidx = (indices.reshape(B, N)
            + (jnp.arange(B, dtype=jnp.int32) * HW)[:, None]).reshape(BN)
    qk_g, v_g = _sc_gather(qk_tok, v_tok, gidx)
    out_tok = _attention(qk_g, v_g, sims.reshape(BN // _RWS, 1, _RWS))
    acc = _sc_scatter(out_tok, gidx)
    return _merge(acc, v_tok)


def kernel(x, sims, mask, ln_w, ln_b, q_w, k_w, v_w, indices, labels,
           num_spixels):
    del mask, labels, num_spixels
    return _run(x, sims, ln_w, ln_b, q_w, k_w, v_w, indices)


# speedup vs baseline: 3.1158x; 1.0614x over previous
import jax
import jax.numpy as jnp
from jax import lax
from jax.experimental import pallas as pl
from jax.experimental.pallas import tpu as pltpu
from jax.experimental.pallas import tpu_sc as plsc

B, C, H, W = 2, 96, 384, 384
QK_DIM = 96
NUM_HEADS = 3
K_SP = 576
TOPK = 64
HEAD_DIM = QK_DIM // NUM_HEADS
SC_SCALE = HEAD_DIM ** (-0.5)
HW = H * W
N = K_SP * TOPK
BN = B * N

NC, NS = 2, 16
NW = NC * NS

_HB1 = 8
_T1 = _HB1 * W


def _k1_body(x_ref, lnw_ref, lnb_ref, qw_ref, kw_ref, vw_ref, qk_ref, v_ref):
    x2 = jnp.reshape(x_ref[0], (C, _T1))
    xt = jnp.transpose(x2, (1, 0))
    mu = jnp.mean(xt, axis=1, keepdims=True)
    var = jnp.mean((xt - mu) ** 2, axis=1, keepdims=True)
    xn = (xt - mu) * lax.rsqrt(var + 1e-6)
    xn = xn * lnw_ref[:] + lnb_ref[:]
    dn = (((1,), (1,)), ((), ()))
    q = lax.dot_general(xn, qw_ref[:], dn, preferred_element_type=jnp.float32)
    k = lax.dot_general(xn, kw_ref[:], dn, preferred_element_type=jnp.float32)
    v = lax.dot_general(xn, vw_ref[:], dn, preferred_element_type=jnp.float32)
    z32 = jnp.zeros((_T1, 32), jnp.float32)
    qk_ref[...] = jnp.concatenate([q, z32, k, z32], axis=1)
    v_ref[...] = jnp.concatenate([v, z32], axis=1)


def _ln_qkv(x, ln_w2, ln_b2, q_w, k_w, v_w):
    grid = (B, H // _HB1)
    return pl.pallas_call(
        _k1_body,
        grid=grid,
        in_specs=[
            pl.BlockSpec((1, C, _HB1, W), lambda b, t: (b, 0, t, 0)),
            pl.BlockSpec((1, C), lambda b, t: (0, 0)),
            pl.BlockSpec((1, C), lambda b, t: (0, 0)),
            pl.BlockSpec((C, C), lambda b, t: (0, 0)),
            pl.BlockSpec((C, C), lambda b, t: (0, 0)),
            pl.BlockSpec((C, C), lambda b, t: (0, 0)),
        ],
        out_specs=[
            pl.BlockSpec((_T1, 256), lambda b, t: (b * (H // _HB1) + t, 0)),
            pl.BlockSpec((_T1, 128), lambda b, t: (b * (H // _HB1) + t, 0)),
        ],
        out_shape=[
            jax.ShapeDtypeStruct((B * HW, 256), jnp.float32),
            jax.ShapeDtypeStruct((B * HW, 128), jnp.float32),
        ],
        compiler_params=pltpu.CompilerParams(
            dimension_semantics=("parallel", "parallel")),
    )(x, ln_w2, ln_b2, q_w, k_w, v_w)


_G_ROWS = BN // NW
_G_CH = 96


def _k2_body(qk_hbm, v_hbm, gidx_hbm, qkg_hbm, vg_hbm, idx_v, qbuf, vbuf, sem):
    wid = lax.axis_index("s") * NC + lax.axis_index("c")
    base = wid * _G_ROWS
    pltpu.sync_copy(gidx_hbm.at[pl.ds(base, _G_ROWS)], idx_v)

    def chunk(i, carry):
        off = i * _G_CH
        pltpu.async_copy(qk_hbm.at[idx_v.at[pl.ds(off, _G_CH)]], qbuf, sem).wait()
        pltpu.sync_copy(qbuf, qkg_hbm.at[pl.ds(base + off, _G_CH)])
        pltpu.async_copy(v_hbm.at[idx_v.at[pl.ds(off, _G_CH)]], vbuf, sem).wait()
        pltpu.sync_copy(vbuf, vg_hbm.at[pl.ds(base + off, _G_CH)])
        return carry

    lax.fori_loop(0, _G_ROWS // _G_CH, chunk, 0)


def _sc_gather(qk_tab, v_tab, gidx):
    mesh = plsc.VectorSubcoreMesh(core_axis_name="c", subcore_axis_name="s")
    f = pl.kernel(
        _k2_body,
        out_type=[
            jax.ShapeDtypeStruct((BN, 256), jnp.float32),
            jax.ShapeDtypeStruct((BN, 128), jnp.float32),
        ],
        mesh=mesh,
        scratch_types=[
            pltpu.VMEM((_G_ROWS,), jnp.int32),
            pltpu.VMEM((_G_CH, 256), jnp.float32),
            pltpu.VMEM((_G_CH, 128), jnp.float32),
            pltpu.SemaphoreType.DMA,
        ],
        compiler_params=pltpu.CompilerParams(use_tc_tiling_on_sc=True),
    )
    return f(qk_tab, v_tab, gidx)


_GSP = 4
_RWS = _GSP * TOPK


def _k3_body(qk_ref, v_ref, sims_ref, out_ref, pen_sc):
    X = qk_ref[...]
    XV = v_ref[...]
    sT = jnp.transpose(sims_ref[0], (1, 0))
    ones1 = jnp.ones((_RWS, 1), jnp.float32)

    @pl.when(pl.program_id(0) == 0)
    def _():
        rb = lax.shift_right_logical(
            lax.broadcasted_iota(jnp.int32, (_RWS, _RWS), 0), 6)
        cb = lax.shift_right_logical(
            lax.broadcasted_iota(jnp.int32, (_RWS, _RWS), 1), 6)
        pen_sc[...] = jnp.where(rb == cb, 0.0, -1e30)

    pen = pen_sc[...]
    outs = []
    dn = (((1,), (1,)), ((), ()))
    for h in range(NUM_HEADS):
        q = X[:, h * HEAD_DIM:(h + 1) * HEAD_DIM]
        k = X[:, 128 + h * HEAD_DIM:128 + (h + 1) * HEAD_DIM]
        v = XV[:, h * HEAD_DIM:(h + 1) * HEAD_DIM]
        qn = jnp.sum(q * q, axis=1, keepdims=True)
        kn = jnp.sum(k * k, axis=1, keepdims=True)
        qa = jnp.concatenate([-2.0 * q, ones1], axis=1)
        ka = jnp.concatenate([k, kn], axis=1)
        d2k = lax.dot_general(qa, ka, dn,
                              preferred_element_type=jnp.float32)
        d2 = d2k + qn
        dist = jnp.sqrt(jnp.maximum(d2, 1e-12))
        e = jnp.exp(pen - SC_SCALE * dist)
        vwa = jnp.concatenate([v * sT, ones1], axis=1)
        dn2 = (((1,), (0,)), ((), ()))
        os = lax.dot_general(e, vwa, dn2,
                             preferred_element_type=jnp.float32)
        outs.append(os[:, 0:HEAD_DIM] * (sT / os[:, HEAD_DIM:HEAD_DIM + 1]))
    z31 = jnp.zeros((_RWS, 31), jnp.float32)
    out_ref[...] = jnp.concatenate(outs + [ones1, z31], axis=1)


def _attention(qk_g, v_g, sims2):
    grid = (BN // _RWS,)
    return pl.pallas_call(
        _k3_body,
        grid=grid,
        in_specs=[
            pl.BlockSpec((_RWS, 256), lambda i: (i, 0)),
            pl.BlockSpec((_RWS, 128), lambda i: (i, 0)),
            pl.BlockSpec((1, 1, _RWS), lambda i: (i, 0, 0)),
        ],
        out_specs=pl.BlockSpec((_RWS, 128), lambda i: (i, 0)),
        out_shape=jax.ShapeDtypeStruct((BN, 128), jnp.float32),
        scratch_shapes=[pltpu.VMEM((_RWS, _RWS), jnp.float32)],
        compiler_params=pltpu.CompilerParams(
            dimension_semantics=("arbitrary",)),
    )(qk_g, v_g, sims2)


_R = 12288
_NRANGE = HW // _R
_S_TOK = N // NS
_S_CH = 128
_ZROWS = 32


def _k4_body(tok_hbm, gidx_hbm, acc_hbm,
             idxa_v, cidx, cdst, cidx_c, cdst_c, bufa, zbuf,
             sema, semz, acc_sp):
    c = lax.axis_index("c")
    s = lax.axis_index("s")
    base_tok = c * N + s * _S_TOK

    with jax.named_scope("k4_init"):
        def zfill(i, carry):
            def zf2(j, carry2):
                zbuf[i, pl.ds(j * 16, 16)] = jnp.zeros((16,), jnp.float32)
                return carry2
            lax.fori_loop(0, 128 // 16, zf2, 0)
            return carry
        lax.fori_loop(0, _ZROWS, zfill, 0)

    pltpu.sync_copy(gidx_hbm.at[pl.ds(base_tok, _S_TOK)], idxa_v)
    iota16 = lax.iota(jnp.int32, 16)

    def one_range(rr, carry):
        r0g = c * HW + rr * _R
        with jax.named_scope("k4_zero"):
            def za(i, carry2):
                pltpu.async_copy(
                    zbuf,
                    acc_sp.at[pl.ds(s * (_R // NS) + i * _ZROWS, _ZROWS)],
                    semz)
                return carry2
            lax.fori_loop(0, _R // NS // _ZROWS, za, 0)

        with jax.named_scope("k4_compact"):
            def cp(i, off):
                vv = idxa_v[pl.ds(i * 16, 16)] - r0g
                msk = (vv >= 0) & (vv < _R)
                rows = base_tok + i * 16 + iota16
                plsc.store_compressed(cidx.at[pl.ds(off, 16)], rows, mask=msk)
                plsc.store_compressed(cdst.at[pl.ds(off, 16)], vv, mask=msk)
                pc = plsc.all_reduce_population_count(msk)
                return off + pc[0]
            nc = lax.fori_loop(0, _S_TOK // 16, cp, 0)
            def pad(k, carry2):
                cidx[pl.ds(nc + k * 16, 16)] = jnp.full((16,), base_tok,
                                                        jnp.int32)
                cdst[pl.ds(nc + k * 16, 16)] = jnp.full((16,), _R, jnp.int32)
                return carry2
            lax.fori_loop(0, _S_CH // 16, pad, 0)

        with jax.named_scope("k4_zdrain"):
            def zd(i, carry2):
                pltpu.make_async_copy(
                    zbuf, acc_sp.at[pl.ds(s * (_R // NS), _ZROWS)],
                    semz).wait()
                return carry2
            lax.fori_loop(0, _R // NS // _ZROWS, zd, 0)
        plsc.subcore_barrier()

        with jax.named_scope("k4_chunks"):
            nch = lax.shift_right_logical(nc + (_S_CH - 1), 7)

            def chunk(j, carry2):
                def mv(k, carry3):
                    cidx_c[pl.ds(k * 16, 16)] = (
                        cidx[pl.ds(j * _S_CH + k * 16, 16)])
                    cdst_c[pl.ds(k * 16, 16)] = (
                        cdst[pl.ds(j * _S_CH + k * 16, 16)])
                    return carry3
                lax.fori_loop(0, _S_CH // 16, mv, 0)
                pltpu.async_copy(tok_hbm.at[cidx_c], bufa, sema).wait()
                pltpu.sync_copy(bufa, acc_sp.at[cdst_c], add=True)
                return carry2
            lax.fori_loop(0, nch, chunk, 0)
        plsc.subcore_barrier()

        with jax.named_scope("k4_out"):
            pltpu.sync_copy(acc_sp.at[pl.ds(s * (_R // NS), _R // NS)],
                            acc_hbm.at[pl.ds(r0g + s * (_R // NS), _R // NS)])
        plsc.subcore_barrier()
        return carry

    lax.fori_loop(0, _NRANGE, one_range, 0)


def _sc_scatter(out_tok, gidx):
    mesh = plsc.VectorSubcoreMesh(core_axis_name="c", subcore_axis_name="s")
    f = pl.kernel(
        _k4_body,
        out_type=jax.ShapeDtypeStruct((B * HW, 128), jnp.float32),
        mesh=mesh,
        scratch_types=[
            pltpu.VMEM((_S_TOK,), jnp.int32),
            pltpu.VMEM((_S_TOK + _S_CH,), jnp.int32),
            pltpu.VMEM((_S_TOK + _S_CH,), jnp.int32),
            pltpu.VMEM((_S_CH,), jnp.int32),
            pltpu.VMEM((_S_CH,), jnp.int32),
            pltpu.VMEM((_S_CH, 128), jnp.float32),
            pltpu.VMEM((_ZROWS, 128), jnp.float32),
            pltpu.SemaphoreType.DMA,
            pltpu.SemaphoreType.DMA,
            pltpu.VMEM_SHARED((_R + 16, 128), jnp.float32),
        ],
        compiler_params=pltpu.CompilerParams(use_tc_tiling_on_sc=True,
                                             needs_layout_passes=False),
    )
    return f(out_tok, gidx)


_T5 = 12288
_HB5 = _T5 // W


def _k5_body(acc_ref, v_ref, out_ref):
    a = acc_ref[...]
    ct = a[:, C:C + 1]
    v = v_ref[:, 0:C]
    mean = a[:, 0:C] / jnp.maximum(ct, 1.0)
    res = jnp.where(ct > 1e-5, mean, v)
    rT = jnp.transpose(res, (1, 0))
    for hb in range(_HB5):
        out_ref[0, :, hb, :] = rT[:, hb * W:(hb + 1) * W]


def _merge(acc, v_tok):
    grid = (B * HW // _T5,)
    nh = H // _HB5
    return pl.pallas_call(
        _k5_body,
        grid=grid,
        in_specs=[
            pl.BlockSpec((_T5, 128), lambda t: (t, 0)),
            pl.BlockSpec((_T5, 128), lambda t: (t, 0)),
        ],
        out_specs=pl.BlockSpec((1, C, _HB5, W), lambda t: (t // nh, 0, t % nh, 0)),
        out_shape=jax.ShapeDtypeStruct((B, C, H, W), jnp.float32),
        compiler_params=pltpu.CompilerParams(
            dimension_semantics=("arbitrary",)),
    )(acc, v_tok)


@jax.jit
def _run(x, sims, ln_w, ln_b, q_w, k_w, v_w, indices):
    qk_tok, v_tok = _ln_qkv(x, ln_w.reshape(1, C), ln_b.reshape(1, C),
                            q_w, k_w, v_w)
    gidx = (indices.reshape(B, N)
            + (jnp.arange(B, dtype=jnp.int32) * HW)[:, None]).reshape(BN)
    qk_g, v_g = _sc_gather(qk_tok, v_tok, gidx)
    out_tok = _attention(qk_g, v_g, sims.reshape(BN // _RWS, 1, _RWS))
    acc = _sc_scatter(out_tok, gidx)
    return _merge(acc, v_tok)


def kernel(x, sims, mask, ln_w, ln_b, q_w, k_w, v_w, indices, labels,
           num_spixels):
    del mask, labels, num_spixels
    return _run(x, sims, ln_w, ln_b, q_w, k_w, v_w, indices)
